# Initial kernel scaffold; baseline (speedup 1.0000x reference)
import jax
import jax.numpy as jnp
from jax.experimental import pallas as pl


def kernel(x, edge_index, edge_attr, Wp, bp, W0, as0, ad0, ae0, We0, b0, g0, be0, W1, as1, ad1, ae1, We1, b1, g1, be1):
    raise NotImplementedError("write your pallas kernel here")


# speedup vs baseline: 46.6769x; 46.6769x over previous
import functools

import jax
import jax.numpy as jnp
from jax import lax
from jax.experimental import pallas as pl
from jax.experimental.pallas import tpu as pltpu
from jax.experimental.pallas import tpu_sc as plsc

N = 10000
E = 320000
DIN = 3
D = 128
H = 4
C = 32
ED = 4

TS = 144
TD = 16
NC = 2
NS = 16
NW = NC * NS
EW = E // NW
SUB = 80
KSUB = 1
BLK = SUB * KSUB
NBLK = EW // BLK
NROW = 624
NTAIL = N - NROW * NS


def _proj_body(x_ref, wp_ref, bp_ref, o_ref):
    o_ref[...] = jnp.dot(x_ref[...], wp_ref[...],
                         preferred_element_type=jnp.float32) + bp_ref[...]


def _proj(x, Wp, bp):
    BN = 2000
    return pl.pallas_call(
        _proj_body,
        grid=(N // BN,),
        in_specs=[
            pl.BlockSpec((BN, DIN), lambda i: (i, 0)),
            pl.BlockSpec((DIN, D), lambda i: (0, 0)),
            pl.BlockSpec((1, D), lambda i: (0, 0)),
        ],
        out_specs=pl.BlockSpec((BN, D), lambda i: (i, 0)),
        out_shape=jax.ShapeDtypeStruct((N, D), jnp.float32),
    )(x, Wp, bp.reshape(1, D))


def _edge_body(ea_ref, wef_ref, e0_ref, e1_ref, sum_ref, max_ref):
    ea = ea_ref[...]
    p = jnp.dot(ea, wef_ref[...], preferred_element_type=jnp.float32)
    e0_ref[...] = p[:, :H]
    e1_ref[...] = p[:, H:]
    bsum = jnp.sum(ea, axis=0, keepdims=True)
    bmax = jnp.max(p, axis=0, keepdims=True)

    @pl.when(pl.program_id(0) == 0)
    def _():
        sum_ref[...] = bsum
        max_ref[...] = bmax

    @pl.when(pl.program_id(0) != 0)
    def _():
        sum_ref[...] += bsum
        max_ref[...] = jnp.maximum(max_ref[...], bmax)


def _edge_feats(ea, Wef01):
    BE = 8000
    return pl.pallas_call(
        _edge_body,
        grid=(E // BE,),
        in_specs=[
            pl.BlockSpec((BE, ED), lambda i: (i, 0)),
            pl.BlockSpec((ED, 2 * H), lambda i: (0, 0)),
        ],
        out_specs=[
            pl.BlockSpec((BE, H), lambda i: (i, 0)),
            pl.BlockSpec((BE, H), lambda i: (i, 0)),
            pl.BlockSpec((1, ED), lambda i: (0, 0)),
            pl.BlockSpec((1, 2 * H), lambda i: (0, 0)),
        ],
        out_shape=[
            jax.ShapeDtypeStruct((E, H), jnp.float32),
            jax.ShapeDtypeStruct((E, H), jnp.float32),
            jax.ShapeDtypeStruct((1, ED), jnp.float32),
            jax.ShapeDtypeStruct((1, 2 * H), jnp.float32),
        ],
    )(ea, Wef01)


def _node_body(h_ref, w_ref, ts_ref, ad_ref, amax_ref):
    t = jnp.dot(h_ref[...], w_ref[...], preferred_element_type=jnp.float32)
    bn = t.shape[0]
    ts_ref[...] = jnp.concatenate(
        [t[:, :D + H], jnp.zeros((bn, TS - D - H), jnp.float32)], axis=1)
    ad_ref[...] = t[:, D:D + 2 * H]
    bmax = jnp.max(t[:, D:D + H], axis=0, keepdims=True)

    @pl.when(pl.program_id(0) == 0)
    def _():
        amax_ref[...] = bmax

    @pl.when(pl.program_id(0) != 0)
    def _():
        amax_ref[...] = jnp.maximum(amax_ref[...], bmax)


def _node_feats(h, Wcat):
    BN = 2000
    return pl.pallas_call(
        _node_body,
        grid=(N // BN,),
        in_specs=[
            pl.BlockSpec((BN, D), lambda i: (i, 0)),
            pl.BlockSpec((D, D + 2 * H), lambda i: (0, 0)),
        ],
        out_specs=[
            pl.BlockSpec((BN, TS), lambda i: (i, 0)),
            pl.BlockSpec((BN, 2 * H), lambda i: (i, 0)),
            pl.BlockSpec((1, H), lambda i: (0, 0)),
        ],
        out_shape=[
            jax.ShapeDtypeStruct((N, TS), jnp.float32),
            jax.ShapeDtypeStruct((N, 2 * H), jnp.float32),
            jax.ShapeDtypeStruct((1, H), jnp.float32),
        ],
    )(h, Wcat)


def _dst_body(ad_ref, cst_ref, td_ref):
    ad = ad_ref[...]
    asrc = ad[:, :H]
    adst = ad[:, H:]
    ame = cst_ref[0, :H]
    eloop = cst_ref[0, H:2 * H]
    zb = adst + ame[None, :]
    bnd = jnp.maximum(zb, 0.2 * zb)
    zs = asrc + adst + eloop[None, :]
    zs = jnp.maximum(zs, 0.2 * zs)
    exs = jnp.exp(zs - bnd)
    bn = ad.shape[0]
    td_ref[...] = jnp.concatenate(
        [adst, bnd, exs, jnp.zeros((bn, TD - 3 * H), jnp.float32)], axis=1)


def _dst_table(AD, cst):
    BN = 2000
    return pl.pallas_call(
        _dst_body,
        grid=(N // BN,),
        in_specs=[
            pl.BlockSpec((BN, 2 * H), lambda i: (i, 0)),
            pl.BlockSpec((1, TD), lambda i: (0, 0)),
        ],
        out_specs=pl.BlockSpec((BN, TD), lambda i: (i, 0)),
        out_shape=jax.ShapeDtypeStruct((N, TD), jnp.float32),
    )(AD, cst)


def _post_body(p0_ref, p1_ref, ts_ref, td_ref, res_ref, prm_ref, o_ref):
    rep = jnp.repeat(jnp.eye(H, dtype=jnp.float32), C, axis=1)
    xs = ts_ref[:, :D]
    exs = td_ref[:, 2 * H:3 * H]
    raw = (p0_ref[0, :, :D] + p1_ref[0, :, :D]
           + xs * jnp.dot(exs, rep, preferred_element_type=jnp.float32))
    den = p0_ref[0, :, D:D + H] + p1_ref[0, :, D:D + H] + exs + 1e-16
    o = raw * jnp.dot(1.0 / den, rep, preferred_element_type=jnp.float32)
    o = o + prm_ref[0, :][None, :]
    mu = jnp.mean(o, axis=-1, keepdims=True)
    oc = o - mu
    var = jnp.mean(oc * oc, axis=-1, keepdims=True)
    o = oc * jax.lax.rsqrt(var + 1e-5) * prm_ref[1, :][None, :] + prm_ref[2, :][None, :]
    o = jnp.where(o > 0, o, jnp.exp(o) - 1.0)
    o_ref[...] = o + res_ref[...]


def _post(parts, Tsrc, Tdst, res, prm):
    BN = 2000
    return pl.pallas_call(
        _post_body,
        grid=(N // BN,),
        in_specs=[
            pl.BlockSpec((1, BN, TS), lambda i: (0, i, 0)),
            pl.BlockSpec((1, BN, TS), lambda i: (1, i, 0)),
            pl.BlockSpec((BN, TS), lambda i: (i, 0)),
            pl.BlockSpec((BN, TD), lambda i: (i, 0)),
            pl.BlockSpec((BN, D), lambda i: (i, 0)),
            pl.BlockSpec((3, D), lambda i: (0, 0)),
        ],
        out_specs=pl.BlockSpec((BN, D), lambda i: (i, 0)),
        out_shape=jax.ShapeDtypeStruct((N, D), jnp.float32),
    )(parts, parts, Tsrc, Tdst, res, prm)


def _sc_edge_pass(tsrc, tdst, eatt, src2, dst2, zrows):
    mesh = plsc.VectorSubcoreMesh(core_axis_name="c", subcore_axis_name="s")

    @functools.partial(
        pl.kernel,
        mesh=mesh,
        out_type=jax.ShapeDtypeStruct((NC, N, TS), jnp.float32),
        scratch_types=[
            pltpu.VMEM_SHARED((N, TS), jnp.float32),
            pltpu.VMEM((KSUB, SUB), jnp.int32),
            pltpu.VMEM((KSUB, SUB), jnp.int32),
            pltpu.VMEM((BLK, H), jnp.float32),
            pltpu.VMEM((BLK, TS), jnp.float32),
            pltpu.VMEM((BLK, TD), jnp.float32),
            pltpu.SemaphoreType.DMA,
            pltpu.SemaphoreType.DMA,
        ],
        compiler_params=pltpu.CompilerParams(use_tc_tiling_on_sc=False,
                                             needs_layout_passes=False),
    )
    def k(ts_hbm, td_hbm, ea_hbm, s2_hbm, d2_hbm, z_hbm, out_hbm,
          acc_sh, idx_s, idx_d, ea_v, rows_v, drows_v, gsem, dsem):
        cid = lax.axis_index("c")
        sid = lax.axis_index("s")
        wid = cid * NS + sid

        pltpu.sync_copy(z_hbm.at[pl.ds(0, NROW)],
                        acc_sh.at[pl.ds(sid * NROW, NROW)])

        @pl.when(sid == NS - 1)
        def _():
            pltpu.sync_copy(z_hbm.at[pl.ds(0, NTAIL)],
                            acc_sh.at[pl.ds(NS * NROW, NTAIL)])

        plsc.subcore_barrier()

        def block_body(j, carry):
            bix = wid * NBLK + j
            pltpu.sync_copy(s2_hbm.at[bix], idx_s)
            pltpu.sync_copy(d2_hbm.at[bix], idx_d)
            pltpu.sync_copy(ea_hbm.at[pl.ds(bix * BLK, BLK)], ea_v)
            cps = []
            for kk in range(KSUB):
                cps.append(pltpu.async_copy(
                    ts_hbm.at[idx_s.at[kk]],
                    rows_v.at[pl.ds(kk * SUB, SUB)], gsem))
                cps.append(pltpu.async_copy(
                    td_hbm.at[idx_d.at[kk]],
                    drows_v.at[pl.ds(kk * SUB, SUB)], dsem))
            for cp in cps:
                cp.wait()

            def group_body(g, c2):
                rg = g * 16
                rows = rg + lax.iota(jnp.int32, 16)
                for h in range(H):
                    colA = jnp.full((16,), D + h, jnp.int32)
                    a = plsc.load_gather(rows_v, [rows, colA])
                    cD = plsc.load_gather(
                        drows_v, [rows, jnp.full((16,), h, jnp.int32)])
                    bD = plsc.load_gather(
                        drows_v, [rows, jnp.full((16,), H + h, jnp.int32)])
                    eA = plsc.load_gather(
                        ea_v, [rows, jnp.full((16,), h, jnp.int32)])
                    z = a + cD + eA
                    z = jnp.maximum(z, 0.2 * z)
                    ex = jnp.exp(z - bD)
                    plsc.store_scatter(rows_v, [rows, colA], ex)
                for i in range(16):
                    r = rg + i
                    exv = rows_v[r, pl.ds(D, 16)]
                    for h in range(H):
                        s = exv[h]
                        c0 = h * C
                        rows_v[r, pl.ds(c0, 16)] = rows_v[r, pl.ds(c0, 16)] * s
                        rows_v[r, pl.ds(c0 + 16, 16)] = (
                            rows_v[r, pl.ds(c0 + 16, 16)] * s)
                return c2

            lax.fori_loop(0, BLK // 16, group_body, 0)

            for kk in range(KSUB):
                pltpu.sync_copy(rows_v.at[pl.ds(kk * SUB, SUB)],
                                acc_sh.at[idx_d.at[kk]], add=True)
            return carry

        lax.fori_loop(0, NBLK, block_body, 0)
        plsc.subcore_barrier()
        pltpu.sync_copy(acc_sh.at[pl.ds(sid * NROW, NROW)],
                        out_hbm.at[cid, pl.ds(sid * NROW, NROW)])

        @pl.when(sid == NS - 1)
        def _():
            pltpu.sync_copy(acc_sh.at[pl.ds(NS * NROW, NTAIL)],
                            out_hbm.at[cid, pl.ds(NS * NROW, NTAIL)])

    return k(tsrc, tdst, eatt, src2, dst2, zrows)


def kernel(x, edge_index, edge_attr, Wp, bp, W0, as0, ad0, ae0, We0, b0, g0,
           be0, W1, as1, ad1, ae1, We1, b1, g1, be1):
    f32 = jnp.float32
    src2 = edge_index[0].reshape(E // BLK, KSUB, SUB)
    dst2 = edge_index[1].reshape(E // BLK, KSUB, SUB)
    zrows = jnp.zeros((NROW, TS), f32)

    def fold(Wf, av):
        return jnp.einsum('dhc,hc->dh', Wf.reshape(-1, H, C), av)

    Wef0 = fold(We0, ae0)
    Wef1 = fold(We1, ae1)
    Wef01 = jnp.concatenate([Wef0, Wef1], axis=1)
    Wcat0 = jnp.concatenate([W0, fold(W0, as0), fold(W0, ad0)], axis=1)
    Wcat1 = jnp.concatenate([W1, fold(W1, as1), fold(W1, ad1)], axis=1)

    eatt0, eatt1, easum, emax = _edge_feats(edge_attr, Wef01)
    eamean = easum[0] / E
    eloop0 = eamean @ Wef0
    eloop1 = eamean @ Wef1
    Emax0 = jnp.maximum(emax[0, :H], eloop0)
    Emax1 = jnp.maximum(emax[0, H:], eloop1)

    h = _proj(x, Wp, bp)

    layers = [
        (Wcat0, eatt0, eloop0, Emax0, b0, g0, be0),
        (Wcat1, eatt1, eloop1, Emax1, b1, g1, be1),
    ]
    for (Wcat, eatt, eloop, Emax, b, g, be) in layers:
        Tsrc, AD, amax = _node_feats(h, Wcat)
        cst = jnp.concatenate(
            [amax[0] + Emax, eloop, jnp.zeros((TD - 2 * H,), f32)]
        ).reshape(1, TD)
        Tdst = _dst_table(AD, cst)
        parts = _sc_edge_pass(Tsrc, Tdst, eatt, src2, dst2, zrows)
        prm = jnp.stack([b, g, be])
        h = _post(parts, Tsrc, Tdst, h, prm)
    return h


# speedup vs baseline: 60.8524x; 1.3037x over previous
import functools

import jax
import jax.numpy as jnp
from jax import lax
from jax.experimental import pallas as pl
from jax.experimental.pallas import tpu as pltpu
from jax.experimental.pallas import tpu_sc as plsc

N = 10000
E = 320000
DIN = 3
D = 128
H = 4
C = 32
ED = 4

TS = 144
TD = 16
NC = 2
NS = 16
NW = NC * NS
EW = E // NW
SUB = 80
KSUB = 1
BLK = SUB * KSUB
NBLK = EW // BLK
NROW = 624
NTAIL = N - NROW * NS


def _proj_body(x_ref, wp_ref, bp_ref, o_ref):
    o_ref[...] = jnp.dot(x_ref[...], wp_ref[...],
                         preferred_element_type=jnp.float32) + bp_ref[...]


def _proj(x, Wp, bp):
    BN = 2000
    return pl.pallas_call(
        _proj_body,
        grid=(N // BN,),
        in_specs=[
            pl.BlockSpec((BN, DIN), lambda i: (i, 0)),
            pl.BlockSpec((DIN, D), lambda i: (0, 0)),
            pl.BlockSpec((1, D), lambda i: (0, 0)),
        ],
        out_specs=pl.BlockSpec((BN, D), lambda i: (i, 0)),
        out_shape=jax.ShapeDtypeStruct((N, D), jnp.float32),
    )(x, Wp, bp.reshape(1, D))


def _edge_body(ea_ref, wef_ref, e0_ref, e1_ref, sum_ref, max_ref):
    ea = ea_ref[...]
    p = jnp.dot(ea, wef_ref[...], preferred_element_type=jnp.float32)
    e0_ref[...] = p[:, :H]
    e1_ref[...] = p[:, H:]
    bsum = jnp.sum(ea, axis=0, keepdims=True)
    bmax = jnp.max(p, axis=0, keepdims=True)

    @pl.when(pl.program_id(0) == 0)
    def _():
        sum_ref[...] = bsum
        max_ref[...] = bmax

    @pl.when(pl.program_id(0) != 0)
    def _():
        sum_ref[...] += bsum
        max_ref[...] = jnp.maximum(max_ref[...], bmax)


def _edge_feats(ea, Wef01):
    BE = 8000
    return pl.pallas_call(
        _edge_body,
        grid=(E // BE,),
        in_specs=[
            pl.BlockSpec((BE, ED), lambda i: (i, 0)),
            pl.BlockSpec((ED, 2 * H), lambda i: (0, 0)),
        ],
        out_specs=[
            pl.BlockSpec((BE, H), lambda i: (i, 0)),
            pl.BlockSpec((BE, H), lambda i: (i, 0)),
            pl.BlockSpec((1, ED), lambda i: (0, 0)),
            pl.BlockSpec((1, 2 * H), lambda i: (0, 0)),
        ],
        out_shape=[
            jax.ShapeDtypeStruct((E, H), jnp.float32),
            jax.ShapeDtypeStruct((E, H), jnp.float32),
            jax.ShapeDtypeStruct((1, ED), jnp.float32),
            jax.ShapeDtypeStruct((1, 2 * H), jnp.float32),
        ],
    )(ea, Wef01)


def _node_body(h_ref, w_ref, ts_ref, ad_ref, amax_ref):
    t = jnp.dot(h_ref[...], w_ref[...], preferred_element_type=jnp.float32)
    bn = t.shape[0]
    ts_ref[...] = jnp.concatenate(
        [t[:, :D + H], jnp.zeros((bn, TS - D - H), jnp.float32)], axis=1)
    ad_ref[...] = t[:, D:D + 2 * H]
    bmax = jnp.max(t[:, D:D + H], axis=0, keepdims=True)

    @pl.when(pl.program_id(0) == 0)
    def _():
        amax_ref[...] = bmax

    @pl.when(pl.program_id(0) != 0)
    def _():
        amax_ref[...] = jnp.maximum(amax_ref[...], bmax)


def _node_feats(h, Wcat):
    BN = 2000
    return pl.pallas_call(
        _node_body,
        grid=(N // BN,),
        in_specs=[
            pl.BlockSpec((BN, D), lambda i: (i, 0)),
            pl.BlockSpec((D, D + 2 * H), lambda i: (0, 0)),
        ],
        out_specs=[
            pl.BlockSpec((BN, TS), lambda i: (i, 0)),
            pl.BlockSpec((BN, 2 * H), lambda i: (i, 0)),
            pl.BlockSpec((1, H), lambda i: (0, 0)),
        ],
        out_shape=[
            jax.ShapeDtypeStruct((N, TS), jnp.float32),
            jax.ShapeDtypeStruct((N, 2 * H), jnp.float32),
            jax.ShapeDtypeStruct((1, H), jnp.float32),
        ],
    )(h, Wcat)


def _dst_body(ad_ref, cst_ref, td_ref):
    ad = ad_ref[...]
    asrc = ad[:, :H]
    adst = ad[:, H:]
    ame = cst_ref[0, :H]
    eloop = cst_ref[0, H:2 * H]
    zb = adst + ame[None, :]
    bnd = jnp.maximum(zb, 0.2 * zb)
    zs = asrc + adst + eloop[None, :]
    zs = jnp.maximum(zs, 0.2 * zs)
    exs = jnp.exp(zs - bnd)
    bn = ad.shape[0]
    td_ref[...] = jnp.concatenate(
        [adst, bnd, exs, jnp.zeros((bn, TD - 3 * H), jnp.float32)], axis=1)


def _dst_table(AD, cst):
    BN = 2000
    return pl.pallas_call(
        _dst_body,
        grid=(N // BN,),
        in_specs=[
            pl.BlockSpec((BN, 2 * H), lambda i: (i, 0)),
            pl.BlockSpec((1, TD), lambda i: (0, 0)),
        ],
        out_specs=pl.BlockSpec((BN, TD), lambda i: (i, 0)),
        out_shape=jax.ShapeDtypeStruct((N, TD), jnp.float32),
    )(AD, cst)


def _post_body(p0_ref, p1_ref, ts_ref, td_ref, res_ref, prm_ref, o_ref):
    rep = jnp.repeat(jnp.eye(H, dtype=jnp.float32), C, axis=1)
    xs = ts_ref[:, :D]
    exs = td_ref[:, 2 * H:3 * H]
    raw = (p0_ref[0, :, :D] + p1_ref[0, :, :D]
           + xs * jnp.dot(exs, rep, preferred_element_type=jnp.float32))
    den = p0_ref[0, :, D:D + H] + p1_ref[0, :, D:D + H] + exs + 1e-16
    o = raw * jnp.dot(1.0 / den, rep, preferred_element_type=jnp.float32)
    o = o + prm_ref[0, :][None, :]
    mu = jnp.mean(o, axis=-1, keepdims=True)
    oc = o - mu
    var = jnp.mean(oc * oc, axis=-1, keepdims=True)
    o = oc * jax.lax.rsqrt(var + 1e-5) * prm_ref[1, :][None, :] + prm_ref[2, :][None, :]
    o = jnp.where(o > 0, o, jnp.exp(o) - 1.0)
    o_ref[...] = o + res_ref[...]


def _post(parts, Tsrc, Tdst, res, prm):
    BN = 2000
    return pl.pallas_call(
        _post_body,
        grid=(N // BN,),
        in_specs=[
            pl.BlockSpec((1, BN, TS), lambda i: (0, i, 0)),
            pl.BlockSpec((1, BN, TS), lambda i: (1, i, 0)),
            pl.BlockSpec((BN, TS), lambda i: (i, 0)),
            pl.BlockSpec((BN, TD), lambda i: (i, 0)),
            pl.BlockSpec((BN, D), lambda i: (i, 0)),
            pl.BlockSpec((3, D), lambda i: (0, 0)),
        ],
        out_specs=pl.BlockSpec((BN, D), lambda i: (i, 0)),
        out_shape=jax.ShapeDtypeStruct((N, D), jnp.float32),
    )(parts, parts, Tsrc, Tdst, res, prm)


def _sc_edge_pass(tsrc, tdst, eatt, src2, dst2, zrows):
    mesh = plsc.VectorSubcoreMesh(core_axis_name="c", subcore_axis_name="s")

    @functools.partial(
        pl.kernel,
        mesh=mesh,
        out_type=jax.ShapeDtypeStruct((NC, N, TS), jnp.float32),
        scratch_types=[
            pltpu.VMEM_SHARED((N, TS), jnp.float32),
            pltpu.VMEM((KSUB, SUB), jnp.int32),
            pltpu.VMEM((KSUB, SUB), jnp.int32),
            pltpu.VMEM((BLK, H), jnp.float32),
            pltpu.VMEM((BLK, TS), jnp.float32),
            pltpu.VMEM((BLK, TD), jnp.float32),
            pltpu.VMEM((KSUB, SUB), jnp.int32),
            pltpu.VMEM((KSUB, SUB), jnp.int32),
            pltpu.VMEM((BLK, H), jnp.float32),
            pltpu.VMEM((BLK, TS), jnp.float32),
            pltpu.VMEM((BLK, TD), jnp.float32),
        ] + [pltpu.SemaphoreType.DMA] * 8,
        compiler_params=pltpu.CompilerParams(use_tc_tiling_on_sc=False,
                                             needs_layout_passes=False),
    )
    def k(ts_hbm, td_hbm, ea_hbm, s2_hbm, d2_hbm, z_hbm, out_hbm,
          acc_sh, is0, id0, ev0, rv0, dv0, is1, id1, ev1, rv1, dv1,
          es0, gs0, ds0, ss0, es1, gs1, ds1, ss1):
        cid = lax.axis_index("c")
        sid = lax.axis_index("s")
        wid = cid * NS + sid
        bufs = ((is0, id0, ev0, rv0, dv0, es0, gs0, ds0, ss0),
                (is1, id1, ev1, rv1, dv1, es1, gs1, ds1, ss1))

        pltpu.sync_copy(z_hbm.at[pl.ds(0, NROW)],
                        acc_sh.at[pl.ds(sid * NROW, NROW)])

        @pl.when(sid == NS - 1)
        def _():
            pltpu.sync_copy(z_hbm.at[pl.ds(0, NTAIL)],
                            acc_sh.at[pl.ds(NS * NROW, NTAIL)])

        plsc.subcore_barrier()

        def issue(j, b):
            isb, idb, evb, rvb, dvb, esb, gsb, dsb, _ = bufs[b]
            bix = wid * NBLK + j
            pltpu.sync_copy(s2_hbm.at[bix], isb)
            pltpu.sync_copy(d2_hbm.at[bix], idb)
            pltpu.async_copy(ea_hbm.at[pl.ds(bix * BLK, BLK)], evb, esb)
            pltpu.async_copy(ts_hbm.at[isb.at[0]], rvb, gsb)
            pltpu.async_copy(td_hbm.at[idb.at[0]], dvb, dsb)

        def waitloads(j, b):
            isb, idb, evb, rvb, dvb, esb, gsb, dsb, _ = bufs[b]
            bix = wid * NBLK + j
            pltpu.make_async_copy(
                ea_hbm.at[pl.ds(bix * BLK, BLK)], evb, esb).wait()
            pltpu.make_async_copy(ts_hbm.at[isb.at[0]], rvb, gsb).wait()
            pltpu.make_async_copy(td_hbm.at[idb.at[0]], dvb, dsb).wait()

        def compute(b):
            _, _, evb, rvb, dvb, _, _, _, _ = bufs[b]

            def group_body(g, c2):
                rg = g * 16
                rows = rg + lax.iota(jnp.int32, 16)
                for h in range(H):
                    colA = jnp.full((16,), D + h, jnp.int32)
                    a = plsc.load_gather(rvb, [rows, colA])
                    cD = plsc.load_gather(
                        dvb, [rows, jnp.full((16,), h, jnp.int32)])
                    bD = plsc.load_gather(
                        dvb, [rows, jnp.full((16,), H + h, jnp.int32)])
                    eA = plsc.load_gather(
                        evb, [rows, jnp.full((16,), h, jnp.int32)])
                    z = a + cD + eA
                    z = jnp.maximum(z, 0.2 * z)
                    ex = jnp.exp(z - bD)
                    plsc.store_scatter(rvb, [rows, colA], ex)
                for i in range(16):
                    r = rg + i
                    exv = rvb[r, pl.ds(D, 16)]
                    for h in range(H):
                        s = exv[h]
                        c0 = h * C
                        rvb[r, pl.ds(c0, 16)] = rvb[r, pl.ds(c0, 16)] * s
                        rvb[r, pl.ds(c0 + 16, 16)] = (
                            rvb[r, pl.ds(c0 + 16, 16)] * s)
                return c2

            lax.fori_loop(0, BLK // 16, group_body, 0)

        def scatter(b):
            _, idb, _, rvb, _, _, _, _, ssb = bufs[b]
            pltpu.async_copy(rvb, acc_sh.at[idb.at[0]], ssb, add=True)

        def waitscatter(b):
            _, idb, _, rvb, _, _, _, _, ssb = bufs[b]
            pltpu.make_async_copy(rvb, acc_sh.at[idb.at[0]], ssb).wait()

        issue(0, 0)

        def pair_body(p, carry):
            j0 = 2 * p
            issue(j0 + 1, 1)
            waitloads(j0, 0)
            compute(0)
            scatter(0)
            waitscatter(0)
            issue(j0 + 2, 0)
            waitloads(j0 + 1, 1)
            compute(1)
            scatter(1)
            waitscatter(1)
            return carry

        lax.fori_loop(0, (NBLK - 1) // 2, pair_body, 0)
        waitloads(NBLK - 1, 0)
        compute(0)
        scatter(0)
        waitscatter(0)
        plsc.subcore_barrier()
        pltpu.sync_copy(acc_sh.at[pl.ds(sid * NROW, NROW)],
                        out_hbm.at[cid, pl.ds(sid * NROW, NROW)])

        @pl.when(sid == NS - 1)
        def _():
            pltpu.sync_copy(acc_sh.at[pl.ds(NS * NROW, NTAIL)],
                            out_hbm.at[cid, pl.ds(NS * NROW, NTAIL)])

    return k(tsrc, tdst, eatt, src2, dst2, zrows)


def kernel(x, edge_index, edge_attr, Wp, bp, W0, as0, ad0, ae0, We0, b0, g0,
           be0, W1, as1, ad1, ae1, We1, b1, g1, be1):
    f32 = jnp.float32
    src2 = edge_index[0].reshape(E // BLK, KSUB, SUB)
    dst2 = edge_index[1].reshape(E // BLK, KSUB, SUB)
    zrows = jnp.zeros((NROW, TS), f32)

    def fold(Wf, av):
        return jnp.einsum('dhc,hc->dh', Wf.reshape(-1, H, C), av)

    Wef0 = fold(We0, ae0)
    Wef1 = fold(We1, ae1)
    Wef01 = jnp.concatenate([Wef0, Wef1], axis=1)
    Wcat0 = jnp.concatenate([W0, fold(W0, as0), fold(W0, ad0)], axis=1)
    Wcat1 = jnp.concatenate([W1, fold(W1, as1), fold(W1, ad1)], axis=1)

    eatt0, eatt1, easum, emax = _edge_feats(edge_attr, Wef01)
    eamean = easum[0] / E
    eloop0 = eamean @ Wef0
    eloop1 = eamean @ Wef1
    Emax0 = jnp.maximum(emax[0, :H], eloop0)
    Emax1 = jnp.maximum(emax[0, H:], eloop1)

    h = _proj(x, Wp, bp)

    layers = [
        (Wcat0, eatt0, eloop0, Emax0, b0, g0, be0),
        (Wcat1, eatt1, eloop1, Emax1, b1, g1, be1),
    ]
    for (Wcat, eatt, eloop, Emax, b, g, be) in layers:
        Tsrc, AD, amax = _node_feats(h, Wcat)
        cst = jnp.concatenate(
            [amax[0] + Emax, eloop, jnp.zeros((TD - 2 * H,), f32)]
        ).reshape(1, TD)
        Tdst = _dst_table(AD, cst)
        parts = _sc_edge_pass(Tsrc, Tdst, eatt, src2, dst2, zrows)
        prm = jnp.stack([b, g, be])
        h = _post(parts, Tsrc, Tdst, h, prm)
    return h


# speedup vs baseline: 74.2889x; 1.2208x over previous
import functools

import jax
import jax.numpy as jnp
from jax import lax
from jax.experimental import pallas as pl
from jax.experimental.pallas import tpu as pltpu
from jax.experimental.pallas import tpu_sc as plsc

N = 10000
E = 320000
DIN = 3
D = 128
H = 4
C = 32
ED = 4

TS = 144
TD = 16
NC = 2
NS = 16
NW = NC * NS
EW = E // NW
SUB = 80
KSUB = 1
SBLK = 25
BLK = SUB * KSUB
NBLK = EW // BLK
NSUP = NBLK // SBLK
NROW = 624
NTAIL = N - NROW * NS


def _proj_body(x_ref, wp_ref, bp_ref, o_ref):
    o_ref[...] = jnp.dot(x_ref[...], wp_ref[...],
                         preferred_element_type=jnp.float32) + bp_ref[...]


def _proj(x, Wp, bp):
    BN = 2000
    return pl.pallas_call(
        _proj_body,
        grid=(N // BN,),
        in_specs=[
            pl.BlockSpec((BN, DIN), lambda i: (i, 0)),
            pl.BlockSpec((DIN, D), lambda i: (0, 0)),
            pl.BlockSpec((1, D), lambda i: (0, 0)),
        ],
        out_specs=pl.BlockSpec((BN, D), lambda i: (i, 0)),
        out_shape=jax.ShapeDtypeStruct((N, D), jnp.float32),
    )(x, Wp, bp.reshape(1, D))


def _edge_body(ea_ref, wef_ref, e0_ref, e1_ref, sum_ref, max_ref):
    ea = ea_ref[...]
    p = jnp.dot(ea, wef_ref[...], preferred_element_type=jnp.float32)
    e0_ref[...] = p[:, :H]
    e1_ref[...] = p[:, H:]
    bsum = jnp.sum(ea, axis=0, keepdims=True)
    bmax = jnp.max(p, axis=0, keepdims=True)

    @pl.when(pl.program_id(0) == 0)
    def _():
        sum_ref[...] = bsum
        max_ref[...] = bmax

    @pl.when(pl.program_id(0) != 0)
    def _():
        sum_ref[...] += bsum
        max_ref[...] = jnp.maximum(max_ref[...], bmax)


def _edge_feats(ea, Wef01):
    BE = 8000
    return pl.pallas_call(
        _edge_body,
        grid=(E // BE,),
        in_specs=[
            pl.BlockSpec((BE, ED), lambda i: (i, 0)),
            pl.BlockSpec((ED, 2 * H), lambda i: (0, 0)),
        ],
        out_specs=[
            pl.BlockSpec((BE, H), lambda i: (i, 0)),
            pl.BlockSpec((BE, H), lambda i: (i, 0)),
            pl.BlockSpec((1, ED), lambda i: (0, 0)),
            pl.BlockSpec((1, 2 * H), lambda i: (0, 0)),
        ],
        out_shape=[
            jax.ShapeDtypeStruct((E, H), jnp.float32),
            jax.ShapeDtypeStruct((E, H), jnp.float32),
            jax.ShapeDtypeStruct((1, ED), jnp.float32),
            jax.ShapeDtypeStruct((1, 2 * H), jnp.float32),
        ],
    )(ea, Wef01)


def _node_body(h_ref, w_ref, ts_ref, ad_ref, amax_ref):
    t = jnp.dot(h_ref[...], w_ref[...], preferred_element_type=jnp.float32)
    bn = t.shape[0]
    ts_ref[...] = jnp.concatenate(
        [t[:, :D + H], jnp.zeros((bn, TS - D - H), jnp.float32)], axis=1)
    ad_ref[...] = t[:, D:D + 2 * H]
    bmax = jnp.max(t[:, D:D + H], axis=0, keepdims=True)

    @pl.when(pl.program_id(0) == 0)
    def _():
        amax_ref[...] = bmax

    @pl.when(pl.program_id(0) != 0)
    def _():
        amax_ref[...] = jnp.maximum(amax_ref[...], bmax)


def _node_feats(h, Wcat):
    BN = 2000
    return pl.pallas_call(
        _node_body,
        grid=(N // BN,),
        in_specs=[
            pl.BlockSpec((BN, D), lambda i: (i, 0)),
            pl.BlockSpec((D, D + 2 * H), lambda i: (0, 0)),
        ],
        out_specs=[
            pl.BlockSpec((BN, TS), lambda i: (i, 0)),
            pl.BlockSpec((BN, 2 * H), lambda i: (i, 0)),
            pl.BlockSpec((1, H), lambda i: (0, 0)),
        ],
        out_shape=[
            jax.ShapeDtypeStruct((N, TS), jnp.float32),
            jax.ShapeDtypeStruct((N, 2 * H), jnp.float32),
            jax.ShapeDtypeStruct((1, H), jnp.float32),
        ],
    )(h, Wcat)


def _dst_body(ad_ref, cst_ref, td_ref):
    ad = ad_ref[...]
    asrc = ad[:, :H]
    adst = ad[:, H:]
    ame = cst_ref[0, :H]
    eloop = cst_ref[0, H:2 * H]
    zb = adst + ame[None, :]
    bnd = jnp.maximum(zb, 0.2 * zb)
    zs = asrc + adst + eloop[None, :]
    zs = jnp.maximum(zs, 0.2 * zs)
    exs = jnp.exp(zs - bnd)
    bn = ad.shape[0]
    td_ref[...] = jnp.concatenate(
        [adst, bnd, exs, jnp.zeros((bn, TD - 3 * H), jnp.float32)], axis=1)


def _dst_table(AD, cst):
    BN = 2000
    return pl.pallas_call(
        _dst_body,
        grid=(N // BN,),
        in_specs=[
            pl.BlockSpec((BN, 2 * H), lambda i: (i, 0)),
            pl.BlockSpec((1, TD), lambda i: (0, 0)),
        ],
        out_specs=pl.BlockSpec((BN, TD), lambda i: (i, 0)),
        out_shape=jax.ShapeDtypeStruct((N, TD), jnp.float32),
    )(AD, cst)


def _post_body(p0_ref, p1_ref, ts_ref, td_ref, res_ref, prm_ref, o_ref):
    rep = jnp.repeat(jnp.eye(H, dtype=jnp.float32), C, axis=1)
    xs = ts_ref[:, :D]
    exs = td_ref[:, 2 * H:3 * H]
    raw = (p0_ref[0, :, :D] + p1_ref[0, :, :D]
           + xs * jnp.dot(exs, rep, preferred_element_type=jnp.float32))
    den = p0_ref[0, :, D:D + H] + p1_ref[0, :, D:D + H] + exs + 1e-16
    o = raw * jnp.dot(1.0 / den, rep, preferred_element_type=jnp.float32)
    o = o + prm_ref[0, :][None, :]
    mu = jnp.mean(o, axis=-1, keepdims=True)
    oc = o - mu
    var = jnp.mean(oc * oc, axis=-1, keepdims=True)
    o = oc * jax.lax.rsqrt(var + 1e-5) * prm_ref[1, :][None, :] + prm_ref[2, :][None, :]
    o = jnp.where(o > 0, o, jnp.exp(o) - 1.0)
    o_ref[...] = o + res_ref[...]


def _post(parts, Tsrc, Tdst, res, prm):
    BN = 2000
    return pl.pallas_call(
        _post_body,
        grid=(N // BN,),
        in_specs=[
            pl.BlockSpec((1, BN, TS), lambda i: (0, i, 0)),
            pl.BlockSpec((1, BN, TS), lambda i: (1, i, 0)),
            pl.BlockSpec((BN, TS), lambda i: (i, 0)),
            pl.BlockSpec((BN, TD), lambda i: (i, 0)),
            pl.BlockSpec((BN, D), lambda i: (i, 0)),
            pl.BlockSpec((3, D), lambda i: (0, 0)),
        ],
        out_specs=pl.BlockSpec((BN, D), lambda i: (i, 0)),
        out_shape=jax.ShapeDtypeStruct((N, D), jnp.float32),
    )(parts, parts, Tsrc, Tdst, res, prm)


def _sc_edge_pass(tsrc, tdst, eatt, src2, dst2, zrows):
    mesh = plsc.VectorSubcoreMesh(core_axis_name="c", subcore_axis_name="s")

    @functools.partial(
        pl.kernel,
        mesh=mesh,
        out_type=jax.ShapeDtypeStruct((NC, N, TS), jnp.float32),
        scratch_types=[
            pltpu.VMEM_SHARED((N, TS), jnp.float32),
            pltpu.VMEM((2, SBLK, SUB), jnp.int32),
            pltpu.VMEM((2, SBLK, SUB), jnp.int32),
            pltpu.VMEM((BLK, H), jnp.float32),
            pltpu.VMEM((BLK, TS), jnp.float32),
            pltpu.VMEM((BLK, TD), jnp.float32),
            pltpu.VMEM((BLK, H), jnp.float32),
            pltpu.VMEM((BLK, TS), jnp.float32),
            pltpu.VMEM((BLK, TD), jnp.float32),
        ] + [pltpu.SemaphoreType.DMA] * 10,
        compiler_params=pltpu.CompilerParams(use_tc_tiling_on_sc=False,
                                             needs_layout_passes=False),
    )
    def k(ts_hbm, td_hbm, ea_hbm, s2_hbm, d2_hbm, z_hbm, out_hbm,
          acc_sh, sidx, didx, ev0, rv0, dv0, ev1, rv1, dv1,
          es0, gs0, ds0, ss0, es1, gs1, ds1, ss1, bs, bd):
        cid = lax.axis_index("c")
        sid = lax.axis_index("s")
        wid = cid * NS + sid
        bufs = ((ev0, rv0, dv0, es0, gs0, ds0, ss0),
                (ev1, rv1, dv1, es1, gs1, ds1, ss1))

        pltpu.sync_copy(z_hbm.at[pl.ds(0, NROW)],
                        acc_sh.at[pl.ds(sid * NROW, NROW)])

        @pl.when(sid == NS - 1)
        def _():
            pltpu.sync_copy(z_hbm.at[pl.ds(0, NTAIL)],
                            acc_sh.at[pl.ds(NS * NROW, NTAIL)])

        plsc.subcore_barrier()

        def bulk_issue(s, sb):
            bsix = wid * NSUP + s
            pltpu.async_copy(s2_hbm.at[bsix], sidx.at[sb], bs)
            pltpu.async_copy(d2_hbm.at[bsix], didx.at[sb], bd)

        def bulk_wait(s, sb):
            bsix = wid * NSUP + s
            pltpu.make_async_copy(s2_hbm.at[bsix], sidx.at[sb], bs).wait()
            pltpu.make_async_copy(d2_hbm.at[bsix], didx.at[sb], bd).wait()

        def issue(s, sb, m, b):
            evb, rvb, dvb, esb, gsb, dsb, _ = bufs[b]
            bix = (wid * NSUP + s) * SBLK + m
            pltpu.async_copy(ea_hbm.at[pl.ds(bix * BLK, BLK)], evb, esb)
            pltpu.async_copy(ts_hbm.at[sidx.at[sb, m]], rvb, gsb)
            pltpu.async_copy(td_hbm.at[didx.at[sb, m]], dvb, dsb)

        def waitloads(s, sb, m, b):
            evb, rvb, dvb, esb, gsb, dsb, _ = bufs[b]
            bix = (wid * NSUP + s) * SBLK + m
            pltpu.make_async_copy(
                ea_hbm.at[pl.ds(bix * BLK, BLK)], evb, esb).wait()
            pltpu.make_async_copy(ts_hbm.at[sidx.at[sb, m]], rvb, gsb).wait()
            pltpu.make_async_copy(td_hbm.at[didx.at[sb, m]], dvb, dsb).wait()

        def compute(b):
            evb, rvb, dvb, _, _, _, _ = bufs[b]

            def group_body(g, c2):
                rg = g * 16
                rows = rg + lax.iota(jnp.int32, 16)
                for h in range(H):
                    colA = jnp.full((16,), D + h, jnp.int32)
                    a = plsc.load_gather(rvb, [rows, colA])
                    cD = plsc.load_gather(
                        dvb, [rows, jnp.full((16,), h, jnp.int32)])
                    bD = plsc.load_gather(
                        dvb, [rows, jnp.full((16,), H + h, jnp.int32)])
                    eA = plsc.load_gather(
                        evb, [rows, jnp.full((16,), h, jnp.int32)])
                    z = a + cD + eA
                    z = jnp.maximum(z, 0.2 * z)
                    ex = jnp.exp(z - bD)
                    plsc.store_scatter(rvb, [rows, colA], ex)
                for i in range(16):
                    r = rg + i
                    exv = rvb[r, pl.ds(D, 16)]
                    for h in range(H):
                        s = exv[h]
                        c0 = h * C
                        rvb[r, pl.ds(c0, 16)] = rvb[r, pl.ds(c0, 16)] * s
                        rvb[r, pl.ds(c0 + 16, 16)] = (
                            rvb[r, pl.ds(c0 + 16, 16)] * s)
                return c2

            lax.fori_loop(0, BLK // 16, group_body, 0)

        def scatter(sb, m, b):
            _, rvb, _, _, _, _, ssb = bufs[b]
            pltpu.async_copy(rvb, acc_sh.at[didx.at[sb, m]], ssb, add=True)

        def waitscatter(sb, m, b):
            _, rvb, _, _, _, _, ssb = bufs[b]
            pltpu.make_async_copy(rvb, acc_sh.at[didx.at[sb, m]], ssb).wait()

        bulk_issue(0, 0)

        def super_body(s, carry):
            sb = s % 2
            bulk_wait(s, sb)

            @pl.when(s < NSUP - 1)
            def _():
                bulk_issue(s + 1, 1 - sb)

            issue(s, sb, 0, 0)

            def pair_body(p, c2):
                m0 = 2 * p
                issue(s, sb, m0 + 1, 1)
                waitloads(s, sb, m0, 0)
                compute(0)
                scatter(sb, m0, 0)
                waitscatter(sb, m0, 0)
                issue(s, sb, m0 + 2, 0)
                waitloads(s, sb, m0 + 1, 1)
                compute(1)
                scatter(sb, m0 + 1, 1)
                waitscatter(sb, m0 + 1, 1)
                return c2

            lax.fori_loop(0, (SBLK - 1) // 2, pair_body, 0)
            waitloads(s, sb, SBLK - 1, 0)
            compute(0)
            scatter(sb, SBLK - 1, 0)
            waitscatter(sb, SBLK - 1, 0)
            return carry

        lax.fori_loop(0, NSUP, super_body, 0)
        plsc.subcore_barrier()
        pltpu.sync_copy(acc_sh.at[pl.ds(sid * NROW, NROW)],
                        out_hbm.at[cid, pl.ds(sid * NROW, NROW)])

        @pl.when(sid == NS - 1)
        def _():
            pltpu.sync_copy(acc_sh.at[pl.ds(NS * NROW, NTAIL)],
                            out_hbm.at[cid, pl.ds(NS * NROW, NTAIL)])

    return k(tsrc, tdst, eatt, src2, dst2, zrows)


def kernel(x, edge_index, edge_attr, Wp, bp, W0, as0, ad0, ae0, We0, b0, g0,
           be0, W1, as1, ad1, ae1, We1, b1, g1, be1):
    f32 = jnp.float32
    src2 = edge_index[0].reshape(E // (SBLK * SUB), SBLK, SUB)
    dst2 = edge_index[1].reshape(E // (SBLK * SUB), SBLK, SUB)
    zrows = jnp.zeros((NROW, TS), f32)

    def fold(Wf, av):
        return jnp.einsum('dhc,hc->dh', Wf.reshape(-1, H, C), av)

    Wef0 = fold(We0, ae0)
    Wef1 = fold(We1, ae1)
    Wef01 = jnp.concatenate([Wef0, Wef1], axis=1)
    Wcat0 = jnp.concatenate([W0, fold(W0, as0), fold(W0, ad0)], axis=1)
    Wcat1 = jnp.concatenate([W1, fold(W1, as1), fold(W1, ad1)], axis=1)

    eatt0, eatt1, easum, emax = _edge_feats(edge_attr, Wef01)
    eamean = easum[0] / E
    eloop0 = eamean @ Wef0
    eloop1 = eamean @ Wef1
    Emax0 = jnp.maximum(emax[0, :H], eloop0)
    Emax1 = jnp.maximum(emax[0, H:], eloop1)

    h = _proj(x, Wp, bp)

    layers = [
        (Wcat0, eatt0, eloop0, Emax0, b0, g0, be0),
        (Wcat1, eatt1, eloop1, Emax1, b1, g1, be1),
    ]
    for (Wcat, eatt, eloop, Emax, b, g, be) in layers:
        Tsrc, AD, amax = _node_feats(h, Wcat)
        cst = jnp.concatenate(
            [amax[0] + Emax, eloop, jnp.zeros((TD - 2 * H,), f32)]
        ).reshape(1, TD)
        Tdst = _dst_table(AD, cst)
        parts = _sc_edge_pass(Tsrc, Tdst, eatt, src2, dst2, zrows)
        prm = jnp.stack([b, g, be])
        h = _post(parts, Tsrc, Tdst, h, prm)
    return h


# speedup vs baseline: 77.2306x; 1.0396x over previous
import functools

import jax
import jax.numpy as jnp
from jax import lax
from jax.experimental import pallas as pl
from jax.experimental.pallas import tpu as pltpu
from jax.experimental.pallas import tpu_sc as plsc

N = 10000
E = 320000
DIN = 3
D = 128
H = 4
C = 32
ED = 4

TS = 144
TD = 16
NC = 2
NS = 16
NW = NC * NS
EW = E // NW
SUB = 80
KSUB = 1
SBLK = 25
BLK = SUB * KSUB
NBLK = EW // BLK
NSUP = NBLK // SBLK
NROW = 624
NTAIL = N - NROW * NS


def _node_core(h, wcat_ref, ts_ref, ad_ref, amax_ref):
    t = jnp.dot(h, wcat_ref[...], preferred_element_type=jnp.float32)
    bn = h.shape[0]
    ts_ref[...] = jnp.concatenate(
        [t[:, :D + H], jnp.zeros((bn, TS - D - H), jnp.float32)], axis=1)
    ad_ref[...] = jnp.concatenate(
        [t[:, D + H:D + 2 * H], t[:, D:D + H],
         jnp.zeros((bn, TD - 2 * H), jnp.float32)], axis=1)
    bmax = jnp.max(t[:, D:D + H], axis=0, keepdims=True)

    @pl.when(pl.program_id(0) == 0)
    def _():
        amax_ref[...] = bmax

    @pl.when(pl.program_id(0) != 0)
    def _():
        amax_ref[...] = jnp.maximum(amax_ref[...], bmax)


def _prep0_body(x_ref, wp_ref, bp_ref, wcat_ref, h_ref, ts_ref, ad_ref,
                amax_ref):
    h = jnp.dot(x_ref[...], wp_ref[...],
                preferred_element_type=jnp.float32) + bp_ref[...]
    h_ref[...] = h
    _node_core(h, wcat_ref, ts_ref, ad_ref, amax_ref)


def _prep0(x, Wp, bp, Wcat):
    BN = 2000
    return pl.pallas_call(
        _prep0_body,
        grid=(N // BN,),
        in_specs=[
            pl.BlockSpec((BN, DIN), lambda i: (i, 0)),
            pl.BlockSpec((DIN, D), lambda i: (0, 0)),
            pl.BlockSpec((1, D), lambda i: (0, 0)),
            pl.BlockSpec((D, D + 2 * H), lambda i: (0, 0)),
        ],
        out_specs=[
            pl.BlockSpec((BN, D), lambda i: (i, 0)),
            pl.BlockSpec((BN, TS), lambda i: (i, 0)),
            pl.BlockSpec((BN, TD), lambda i: (i, 0)),
            pl.BlockSpec((1, H), lambda i: (0, 0)),
        ],
        out_shape=[
            jax.ShapeDtypeStruct((N, D), jnp.float32),
            jax.ShapeDtypeStruct((N, TS), jnp.float32),
            jax.ShapeDtypeStruct((N, TD), jnp.float32),
            jax.ShapeDtypeStruct((1, H), jnp.float32),
        ],
    )(x, Wp, bp.reshape(1, D), Wcat)


def _edge_body(ea_ref, wef_ref, e0_ref, e1_ref, sum_ref, max_ref):
    ea = ea_ref[...]
    p = jnp.dot(ea, wef_ref[...], preferred_element_type=jnp.float32)
    e0_ref[...] = p[:, :H]
    e1_ref[...] = p[:, H:]
    bsum = jnp.sum(ea, axis=0, keepdims=True)
    bmax = jnp.max(p, axis=0, keepdims=True)

    @pl.when(pl.program_id(0) == 0)
    def _():
        sum_ref[...] = bsum
        max_ref[...] = bmax

    @pl.when(pl.program_id(0) != 0)
    def _():
        sum_ref[...] += bsum
        max_ref[...] = jnp.maximum(max_ref[...], bmax)


def _edge_feats(ea, Wef01):
    BE = 8000
    return pl.pallas_call(
        _edge_body,
        grid=(E // BE,),
        in_specs=[
            pl.BlockSpec((BE, ED), lambda i: (i, 0)),
            pl.BlockSpec((ED, 2 * H), lambda i: (0, 0)),
        ],
        out_specs=[
            pl.BlockSpec((BE, H), lambda i: (i, 0)),
            pl.BlockSpec((BE, H), lambda i: (i, 0)),
            pl.BlockSpec((1, ED), lambda i: (0, 0)),
            pl.BlockSpec((1, 2 * H), lambda i: (0, 0)),
        ],
        out_shape=[
            jax.ShapeDtypeStruct((E, H), jnp.float32),
            jax.ShapeDtypeStruct((E, H), jnp.float32),
            jax.ShapeDtypeStruct((1, ED), jnp.float32),
            jax.ShapeDtypeStruct((1, 2 * H), jnp.float32),
        ],
    )(ea, Wef01)


def _post_math(p0_ref, p1_ref, ts_ref, ad_ref, cst_ref, res_ref, prm_ref):
    rep = jnp.repeat(jnp.eye(H, dtype=jnp.float32), C, axis=1)
    xs = ts_ref[:, :D]
    ad = ad_ref[...]
    adst = ad[:, :H]
    asrc = ad[:, H:2 * H]
    ame = cst_ref[0, :H]
    eloop = cst_ref[0, H:2 * H]
    zb = adst + ame[None, :]
    bnd = jnp.maximum(zb, 0.2 * zb)
    zs = asrc + adst + eloop[None, :]
    zs = jnp.maximum(zs, 0.2 * zs)
    exs = jnp.exp(zs - bnd)
    raw = (p0_ref[0, :, :D] + p1_ref[0, :, :D]
           + xs * jnp.dot(exs, rep, preferred_element_type=jnp.float32))
    den = p0_ref[0, :, D:D + H] + p1_ref[0, :, D:D + H] + exs + 1e-16
    o = raw * jnp.dot(1.0 / den, rep, preferred_element_type=jnp.float32)
    o = o + prm_ref[0, :][None, :]
    mu = jnp.mean(o, axis=-1, keepdims=True)
    oc = o - mu
    var = jnp.mean(oc * oc, axis=-1, keepdims=True)
    o = oc * jax.lax.rsqrt(var + 1e-5) * prm_ref[1, :][None, :] + prm_ref[2, :][None, :]
    o = jnp.where(o > 0, o, jnp.exp(o) - 1.0)
    return o + res_ref[...]


def _postprep_body(p0_ref, p1_ref, ts_ref, ad_ref, cst_ref, res_ref, prm_ref,
                   wcat_ref, h_ref, ts1_ref, ad1_ref, amax_ref):
    h = _post_math(p0_ref, p1_ref, ts_ref, ad_ref, cst_ref, res_ref, prm_ref)
    h_ref[...] = h
    _node_core(h, wcat_ref, ts1_ref, ad1_ref, amax_ref)


def _postprep(parts, Tsrc, ADp, cst, res, prm, Wcat):
    BN = 2000
    return pl.pallas_call(
        _postprep_body,
        grid=(N // BN,),
        in_specs=[
            pl.BlockSpec((1, BN, TS), lambda i: (0, i, 0)),
            pl.BlockSpec((1, BN, TS), lambda i: (1, i, 0)),
            pl.BlockSpec((BN, TS), lambda i: (i, 0)),
            pl.BlockSpec((BN, TD), lambda i: (i, 0)),
            pl.BlockSpec((1, TD), lambda i: (0, 0)),
            pl.BlockSpec((BN, D), lambda i: (i, 0)),
            pl.BlockSpec((3, D), lambda i: (0, 0)),
            pl.BlockSpec((D, D + 2 * H), lambda i: (0, 0)),
        ],
        out_specs=[
            pl.BlockSpec((BN, D), lambda i: (i, 0)),
            pl.BlockSpec((BN, TS), lambda i: (i, 0)),
            pl.BlockSpec((BN, TD), lambda i: (i, 0)),
            pl.BlockSpec((1, H), lambda i: (0, 0)),
        ],
        out_shape=[
            jax.ShapeDtypeStruct((N, D), jnp.float32),
            jax.ShapeDtypeStruct((N, TS), jnp.float32),
            jax.ShapeDtypeStruct((N, TD), jnp.float32),
            jax.ShapeDtypeStruct((1, H), jnp.float32),
        ],
    )(parts, parts, Tsrc, ADp, cst, res, prm, Wcat)


def _post_final_body(p0_ref, p1_ref, ts_ref, ad_ref, cst_ref, res_ref,
                     prm_ref, o_ref):
    o_ref[...] = _post_math(p0_ref, p1_ref, ts_ref, ad_ref, cst_ref, res_ref,
                            prm_ref)


def _post(parts, Tsrc, ADp, cst, res, prm):
    BN = 2000
    return pl.pallas_call(
        _post_final_body,
        grid=(N // BN,),
        in_specs=[
            pl.BlockSpec((1, BN, TS), lambda i: (0, i, 0)),
            pl.BlockSpec((1, BN, TS), lambda i: (1, i, 0)),
            pl.BlockSpec((BN, TS), lambda i: (i, 0)),
            pl.BlockSpec((BN, TD), lambda i: (i, 0)),
            pl.BlockSpec((1, TD), lambda i: (0, 0)),
            pl.BlockSpec((BN, D), lambda i: (i, 0)),
            pl.BlockSpec((3, D), lambda i: (0, 0)),
        ],
        out_specs=pl.BlockSpec((BN, D), lambda i: (i, 0)),
        out_shape=jax.ShapeDtypeStruct((N, D), jnp.float32),
    )(parts, parts, Tsrc, ADp, cst, res, prm)


def _sc_edge_pass(tsrc, adp, eatt, src2, dst2, zrows, ame16):
    mesh = plsc.VectorSubcoreMesh(core_axis_name="c", subcore_axis_name="s")

    @functools.partial(
        pl.kernel,
        mesh=mesh,
        out_type=jax.ShapeDtypeStruct((NC, N, TS), jnp.float32),
        scratch_types=[
            pltpu.VMEM_SHARED((N, TS), jnp.float32),
            pltpu.VMEM((2, SBLK, SUB), jnp.int32),
            pltpu.VMEM((2, SBLK, SUB), jnp.int32),
            pltpu.VMEM((BLK, H), jnp.float32),
            pltpu.VMEM((BLK, TS), jnp.float32),
            pltpu.VMEM((BLK, TD), jnp.float32),
            pltpu.VMEM((BLK, H), jnp.float32),
            pltpu.VMEM((BLK, TS), jnp.float32),
            pltpu.VMEM((BLK, TD), jnp.float32),
            pltpu.VMEM((16,), jnp.float32),
        ] + [pltpu.SemaphoreType.DMA] * 10,
        compiler_params=pltpu.CompilerParams(use_tc_tiling_on_sc=False,
                                             needs_layout_passes=False),
    )
    def k(ts_hbm, td_hbm, ea_hbm, s2_hbm, d2_hbm, z_hbm, ame_hbm, out_hbm,
          acc_sh, sidx, didx, ev0, rv0, dv0, ev1, rv1, dv1, amec,
          es0, gs0, ds0, ss0, es1, gs1, ds1, ss1, bs, bd):
        cid = lax.axis_index("c")
        sid = lax.axis_index("s")
        wid = cid * NS + sid
        bufs = ((ev0, rv0, dv0, es0, gs0, ds0, ss0),
                (ev1, rv1, dv1, es1, gs1, ds1, ss1))

        pltpu.sync_copy(ame_hbm, amec)
        pltpu.sync_copy(z_hbm.at[pl.ds(0, NROW)],
                        acc_sh.at[pl.ds(sid * NROW, NROW)])

        @pl.when(sid == NS - 1)
        def _():
            pltpu.sync_copy(z_hbm.at[pl.ds(0, NTAIL)],
                            acc_sh.at[pl.ds(NS * NROW, NTAIL)])

        plsc.subcore_barrier()

        def bulk_issue(s, sb):
            bsix = wid * NSUP + s
            pltpu.async_copy(s2_hbm.at[bsix], sidx.at[sb], bs)
            pltpu.async_copy(d2_hbm.at[bsix], didx.at[sb], bd)

        def bulk_wait(s, sb):
            bsix = wid * NSUP + s
            pltpu.make_async_copy(s2_hbm.at[bsix], sidx.at[sb], bs).wait()
            pltpu.make_async_copy(d2_hbm.at[bsix], didx.at[sb], bd).wait()

        def issue(s, sb, m, b):
            evb, rvb, dvb, esb, gsb, dsb, _ = bufs[b]
            bix = (wid * NSUP + s) * SBLK + m
            pltpu.async_copy(ea_hbm.at[pl.ds(bix * BLK, BLK)], evb, esb)
            pltpu.async_copy(ts_hbm.at[sidx.at[sb, m]], rvb, gsb)
            pltpu.async_copy(td_hbm.at[didx.at[sb, m]], dvb, dsb)

        def waitloads(s, sb, m, b):
            evb, rvb, dvb, esb, gsb, dsb, _ = bufs[b]
            bix = (wid * NSUP + s) * SBLK + m
            pltpu.make_async_copy(
                ea_hbm.at[pl.ds(bix * BLK, BLK)], evb, esb).wait()
            pltpu.make_async_copy(ts_hbm.at[sidx.at[sb, m]], rvb, gsb).wait()
            pltpu.make_async_copy(td_hbm.at[didx.at[sb, m]], dvb, dsb).wait()

        def compute(b):
            evb, rvb, dvb, _, _, _, _ = bufs[b]

            def group_body(g, c2):
                rg = g * 16
                rows = rg + lax.iota(jnp.int32, 16)
                amv = amec[...]
                for h in range(H):
                    colA = jnp.full((16,), D + h, jnp.int32)
                    a = plsc.load_gather(rvb, [rows, colA])
                    cD = plsc.load_gather(
                        dvb, [rows, jnp.full((16,), h, jnp.int32)])
                    eA = plsc.load_gather(
                        evb, [rows, jnp.full((16,), h, jnp.int32)])
                    zb = cD + amv[h]
                    bD = jnp.maximum(zb, 0.2 * zb)
                    z = a + cD + eA
                    z = jnp.maximum(z, 0.2 * z)
                    ex = jnp.exp(z - bD)
                    plsc.store_scatter(rvb, [rows, colA], ex)
                for i in range(16):
                    r = rg + i
                    exv = rvb[r, pl.ds(D, 16)]
                    for h in range(H):
                        s = exv[h]
                        c0 = h * C
                        rvb[r, pl.ds(c0, 16)] = rvb[r, pl.ds(c0, 16)] * s
                        rvb[r, pl.ds(c0 + 16, 16)] = (
                            rvb[r, pl.ds(c0 + 16, 16)] * s)
                return c2

            lax.fori_loop(0, BLK // 16, group_body, 0)

        def scatter(sb, m, b):
            _, rvb, _, _, _, _, ssb = bufs[b]
            pltpu.async_copy(rvb, acc_sh.at[didx.at[sb, m]], ssb, add=True)

        def waitscatter(sb, m, b):
            _, rvb, _, _, _, _, ssb = bufs[b]
            pltpu.make_async_copy(rvb, acc_sh.at[didx.at[sb, m]], ssb).wait()

        bulk_issue(0, 0)

        def super_body(s, carry):
            sb = s % 2
            bulk_wait(s, sb)

            @pl.when(s < NSUP - 1)
            def _():
                bulk_issue(s + 1, 1 - sb)

            issue(s, sb, 0, 0)

            def pair_body(p, c2):
                m0 = 2 * p
                issue(s, sb, m0 + 1, 1)
                waitloads(s, sb, m0, 0)
                compute(0)
                scatter(sb, m0, 0)
                waitscatter(sb, m0, 0)
                issue(s, sb, m0 + 2, 0)
                waitloads(s, sb, m0 + 1, 1)
                compute(1)
                scatter(sb, m0 + 1, 1)
                waitscatter(sb, m0 + 1, 1)
                return c2

            lax.fori_loop(0, (SBLK - 1) // 2, pair_body, 0)
            waitloads(s, sb, SBLK - 1, 0)
            compute(0)
            scatter(sb, SBLK - 1, 0)
            waitscatter(sb, SBLK - 1, 0)
            return carry

        lax.fori_loop(0, NSUP, super_body, 0)
        plsc.subcore_barrier()
        pltpu.sync_copy(acc_sh.at[pl.ds(sid * NROW, NROW)],
                        out_hbm.at[cid, pl.ds(sid * NROW, NROW)])

        @pl.when(sid == NS - 1)
        def _():
            pltpu.sync_copy(acc_sh.at[pl.ds(NS * NROW, NTAIL)],
                            out_hbm.at[cid, pl.ds(NS * NROW, NTAIL)])

    return k(tsrc, adp, eatt, src2, dst2, zrows, ame16)


def kernel(x, edge_index, edge_attr, Wp, bp, W0, as0, ad0, ae0, We0, b0, g0,
           be0, W1, as1, ad1, ae1, We1, b1, g1, be1):
    f32 = jnp.float32
    src2 = edge_index[0].reshape(E // (SBLK * SUB), SBLK, SUB)
    dst2 = edge_index[1].reshape(E // (SBLK * SUB), SBLK, SUB)
    zrows = jnp.zeros((NROW, TS), f32)

    def fold(Wf, av):
        return jnp.einsum('dhc,hc->dh', Wf.reshape(-1, H, C), av)

    Wef0 = fold(We0, ae0)
    Wef1 = fold(We1, ae1)
    Wef01 = jnp.concatenate([Wef0, Wef1], axis=1)
    Wcat0 = jnp.concatenate([W0, fold(W0, as0), fold(W0, ad0)], axis=1)
    Wcat1 = jnp.concatenate([W1, fold(W1, as1), fold(W1, ad1)], axis=1)

    eatt0, eatt1, easum, emax = _edge_feats(edge_attr, Wef01)
    eamean = easum[0] / E
    eloop0 = eamean @ Wef0
    eloop1 = eamean @ Wef1
    Emax0 = jnp.maximum(emax[0, :H], eloop0)
    Emax1 = jnp.maximum(emax[0, H:], eloop1)

    def cst_of(amax, Emax, eloop):
        return jnp.concatenate(
            [amax[0] + Emax, eloop, jnp.zeros((TD - 2 * H,), f32)]
        ).reshape(1, TD)

    h0, Tsrc0, ADp0, amax0 = _prep0(x, Wp, bp, Wcat0)
    cst0 = cst_of(amax0, Emax0, eloop0)
    parts0 = _sc_edge_pass(Tsrc0, ADp0, eatt0, src2, dst2, zrows,
                           cst0[0].reshape(16))
    prm0 = jnp.stack([b0, g0, be0])
    h1, Tsrc1, ADp1, amax1 = _postprep(parts0, Tsrc0, ADp0, cst0, h0, prm0,
                                       Wcat1)
    cst1 = cst_of(amax1, Emax1, eloop1)
    parts1 = _sc_edge_pass(Tsrc1, ADp1, eatt1, src2, dst2, zrows,
                           cst1[0].reshape(16))
    prm1 = jnp.stack([b1, g1, be1])
    return _post(parts1, Tsrc1, ADp1, cst1, h1, prm1)


# speedup vs baseline: 89.1703x; 1.1546x over previous
import functools

import jax
import jax.numpy as jnp
from jax import lax
from jax.experimental import pallas as pl
from jax.experimental.pallas import tpu as pltpu
from jax.experimental.pallas import tpu_sc as plsc

N = 10000
E = 320000
DIN = 3
D = 128
H = 4
C = 32
ED = 4

TS = 144
TD = 16
NC = 2
NS = 16
NW = NC * NS
EW = E // NW
SUB = 80
KSUB = 1
SBLK = 25
BLK = SUB * KSUB
NBLK = EW // BLK
NSUP = NBLK // SBLK
NROW = 624
NTAIL = N - NROW * NS


def _node_core(h, wcat_ref, ts_ref, ad_ref, amax_ref):
    t = jnp.dot(h, wcat_ref[...], preferred_element_type=jnp.float32)
    bn = h.shape[0]
    ts_ref[...] = jnp.concatenate(
        [t[:, :D + H], jnp.zeros((bn, TS - D - H), jnp.float32)], axis=1)
    ad_ref[...] = jnp.concatenate(
        [t[:, D + H:D + 2 * H], t[:, D:D + H],
         jnp.zeros((bn, TD - 2 * H), jnp.float32)], axis=1)
    bmax = jnp.max(t[:, D:D + H], axis=0, keepdims=True)

    @pl.when(pl.program_id(0) == 0)
    def _():
        amax_ref[...] = bmax

    @pl.when(pl.program_id(0) != 0)
    def _():
        amax_ref[...] = jnp.maximum(amax_ref[...], bmax)


def _prep0_body(x_ref, wp_ref, bp_ref, wcat_ref, ts_ref, ad_ref, amax_ref):
    h = jnp.dot(x_ref[...], wp_ref[...],
                preferred_element_type=jnp.float32) + bp_ref[...]
    _node_core(h, wcat_ref, ts_ref, ad_ref, amax_ref)


def _prep0(x, Wp, bp, Wcat):
    BN = 2000
    return pl.pallas_call(
        _prep0_body,
        grid=(N // BN,),
        in_specs=[
            pl.BlockSpec((BN, DIN), lambda i: (i, 0)),
            pl.BlockSpec((DIN, D), lambda i: (0, 0)),
            pl.BlockSpec((1, D), lambda i: (0, 0)),
            pl.BlockSpec((D, D + 2 * H), lambda i: (0, 0)),
        ],
        out_specs=[
            pl.BlockSpec((BN, TS), lambda i: (i, 0)),
            pl.BlockSpec((BN, TD), lambda i: (i, 0)),
            pl.BlockSpec((1, H), lambda i: (0, 0)),
        ],
        out_shape=[
            jax.ShapeDtypeStruct((N, TS), jnp.float32),
            jax.ShapeDtypeStruct((N, TD), jnp.float32),
            jax.ShapeDtypeStruct((1, H), jnp.float32),
        ],
    )(x, Wp, bp.reshape(1, D), Wcat)


def _edge_body(ea_ref, wef_ref, sum_ref, max_ref):
    ea = ea_ref[...]
    p = jnp.dot(ea, wef_ref[...], preferred_element_type=jnp.float32)
    bsum = jnp.sum(ea, axis=0, keepdims=True)
    bmax = jnp.max(p, axis=0, keepdims=True)

    @pl.when(pl.program_id(0) == 0)
    def _():
        sum_ref[...] = bsum
        max_ref[...] = bmax

    @pl.when(pl.program_id(0) != 0)
    def _():
        sum_ref[...] += bsum
        max_ref[...] = jnp.maximum(max_ref[...], bmax)


def _edge_feats(ea, Wef01):
    BE = 8000
    return pl.pallas_call(
        _edge_body,
        grid=(E // BE,),
        in_specs=[
            pl.BlockSpec((BE, ED), lambda i: (i, 0)),
            pl.BlockSpec((ED, 2 * H), lambda i: (0, 0)),
        ],
        out_specs=[
            pl.BlockSpec((1, ED), lambda i: (0, 0)),
            pl.BlockSpec((1, 2 * H), lambda i: (0, 0)),
        ],
        out_shape=[
            jax.ShapeDtypeStruct((1, ED), jnp.float32),
            jax.ShapeDtypeStruct((1, 2 * H), jnp.float32),
        ],
    )(ea, Wef01)


def _post_math(p0_ref, p1_ref, ts_ref, ad_ref, cst_ref, res, prm_ref):
    rep = jnp.repeat(jnp.eye(H, dtype=jnp.float32), C, axis=1)
    xs = ts_ref[:, :D]
    ad = ad_ref[...]
    adst = ad[:, :H]
    asrc = ad[:, H:2 * H]
    ame = cst_ref[0, :H]
    eloop = cst_ref[0, H:2 * H]
    zb = adst + ame[None, :]
    bnd = jnp.maximum(zb, 0.2 * zb)
    zs = asrc + adst + eloop[None, :]
    zs = jnp.maximum(zs, 0.2 * zs)
    exs = jnp.exp(zs - bnd)
    raw = (p0_ref[0, :, :D] + p1_ref[0, :, :D]
           + xs * jnp.dot(exs, rep, preferred_element_type=jnp.float32))
    den = p0_ref[0, :, D:D + H] + p1_ref[0, :, D:D + H] + exs + 1e-16
    o = raw * jnp.dot(1.0 / den, rep, preferred_element_type=jnp.float32)
    o = o + prm_ref[0, :][None, :]
    mu = jnp.mean(o, axis=-1, keepdims=True)
    oc = o - mu
    var = jnp.mean(oc * oc, axis=-1, keepdims=True)
    o = oc * jax.lax.rsqrt(var + 1e-5) * prm_ref[1, :][None, :] + prm_ref[2, :][None, :]
    o = jnp.where(o > 0, o, jnp.exp(o) - 1.0)
    return o + res


def _postprep_body(p0_ref, p1_ref, ts_ref, ad_ref, cst_ref, x_ref, wp_ref,
                   bp_ref, prm_ref, wcat_ref, h_ref, ts1_ref, ad1_ref,
                   amax_ref):
    res = jnp.dot(x_ref[...], wp_ref[...],
                  preferred_element_type=jnp.float32) + bp_ref[...]
    h = _post_math(p0_ref, p1_ref, ts_ref, ad_ref, cst_ref, res, prm_ref)
    h_ref[...] = h
    _node_core(h, wcat_ref, ts1_ref, ad1_ref, amax_ref)


def _postprep(parts, Tsrc, ADp, cst, x, Wp, bp, prm, Wcat):
    BN = 2000
    return pl.pallas_call(
        _postprep_body,
        grid=(N // BN,),
        in_specs=[
            pl.BlockSpec((1, BN, TS), lambda i: (0, i, 0)),
            pl.BlockSpec((1, BN, TS), lambda i: (1, i, 0)),
            pl.BlockSpec((BN, TS), lambda i: (i, 0)),
            pl.BlockSpec((BN, TD), lambda i: (i, 0)),
            pl.BlockSpec((1, TD), lambda i: (0, 0)),
            pl.BlockSpec((BN, DIN), lambda i: (i, 0)),
            pl.BlockSpec((DIN, D), lambda i: (0, 0)),
            pl.BlockSpec((1, D), lambda i: (0, 0)),
            pl.BlockSpec((3, D), lambda i: (0, 0)),
            pl.BlockSpec((D, D + 2 * H), lambda i: (0, 0)),
        ],
        out_specs=[
            pl.BlockSpec((BN, D), lambda i: (i, 0)),
            pl.BlockSpec((BN, TS), lambda i: (i, 0)),
            pl.BlockSpec((BN, TD), lambda i: (i, 0)),
            pl.BlockSpec((1, H), lambda i: (0, 0)),
        ],
        out_shape=[
            jax.ShapeDtypeStruct((N, D), jnp.float32),
            jax.ShapeDtypeStruct((N, TS), jnp.float32),
            jax.ShapeDtypeStruct((N, TD), jnp.float32),
            jax.ShapeDtypeStruct((1, H), jnp.float32),
        ],
    )(parts, parts, Tsrc, ADp, cst, x, Wp, bp.reshape(1, D), prm, Wcat)


def _post_final_body(p0_ref, p1_ref, ts_ref, ad_ref, cst_ref, res_ref,
                     prm_ref, o_ref):
    o_ref[...] = _post_math(p0_ref, p1_ref, ts_ref, ad_ref, cst_ref,
                            res_ref[...], prm_ref)


def _post(parts, Tsrc, ADp, cst, res, prm):
    BN = 2000
    return pl.pallas_call(
        _post_final_body,
        grid=(N // BN,),
        in_specs=[
            pl.BlockSpec((1, BN, TS), lambda i: (0, i, 0)),
            pl.BlockSpec((1, BN, TS), lambda i: (1, i, 0)),
            pl.BlockSpec((BN, TS), lambda i: (i, 0)),
            pl.BlockSpec((BN, TD), lambda i: (i, 0)),
            pl.BlockSpec((1, TD), lambda i: (0, 0)),
            pl.BlockSpec((BN, D), lambda i: (i, 0)),
            pl.BlockSpec((3, D), lambda i: (0, 0)),
        ],
        out_specs=pl.BlockSpec((BN, D), lambda i: (i, 0)),
        out_shape=jax.ShapeDtypeStruct((N, D), jnp.float32),
    )(parts, parts, Tsrc, ADp, cst, res, prm)


def _sc_edge_pass(tsrc, adp, ea, src2, dst2, zrows, cvec):
    mesh = plsc.VectorSubcoreMesh(core_axis_name="c", subcore_axis_name="s")

    @functools.partial(
        pl.kernel,
        mesh=mesh,
        out_type=jax.ShapeDtypeStruct((NC, N, TS), jnp.float32),
        scratch_types=[
            pltpu.VMEM_SHARED((N, TS), jnp.float32),
            pltpu.VMEM((2, SBLK, SUB), jnp.int32),
            pltpu.VMEM((2, SBLK, SUB), jnp.int32),
            pltpu.VMEM((BLK, H), jnp.float32),
            pltpu.VMEM((BLK, TS), jnp.float32),
            pltpu.VMEM((BLK, TD), jnp.float32),
            pltpu.VMEM((BLK, H), jnp.float32),
            pltpu.VMEM((BLK, TS), jnp.float32),
            pltpu.VMEM((BLK, TD), jnp.float32),
            pltpu.VMEM((2, 16), jnp.float32),
        ] + [pltpu.SemaphoreType.DMA] * 10,
        compiler_params=pltpu.CompilerParams(use_tc_tiling_on_sc=False,
                                             needs_layout_passes=False),
    )
    def k(ts_hbm, td_hbm, ea_hbm, s2_hbm, d2_hbm, z_hbm, ame_hbm, out_hbm,
          acc_sh, sidx, didx, ev0, rv0, dv0, ev1, rv1, dv1, amec,
          es0, gs0, ds0, ss0, es1, gs1, ds1, ss1, bs, bd):
        cid = lax.axis_index("c")
        sid = lax.axis_index("s")
        wid = cid * NS + sid
        bufs = ((ev0, rv0, dv0, es0, gs0, ds0, ss0),
                (ev1, rv1, dv1, es1, gs1, ds1, ss1))

        pltpu.sync_copy(ame_hbm, amec)
        pltpu.sync_copy(z_hbm.at[pl.ds(0, NROW)],
                        acc_sh.at[pl.ds(sid * NROW, NROW)])

        @pl.when(sid == NS - 1)
        def _():
            pltpu.sync_copy(z_hbm.at[pl.ds(0, NTAIL)],
                            acc_sh.at[pl.ds(NS * NROW, NTAIL)])

        plsc.subcore_barrier()

        def bulk_issue(s, sb):
            bsix = wid * NSUP + s
            pltpu.async_copy(s2_hbm.at[bsix], sidx.at[sb], bs)
            pltpu.async_copy(d2_hbm.at[bsix], didx.at[sb], bd)

        def bulk_wait(s, sb):
            bsix = wid * NSUP + s
            pltpu.make_async_copy(s2_hbm.at[bsix], sidx.at[sb], bs).wait()
            pltpu.make_async_copy(d2_hbm.at[bsix], didx.at[sb], bd).wait()

        def issue(s, sb, m, b):
            evb, rvb, dvb, esb, gsb, dsb, _ = bufs[b]
            bix = (wid * NSUP + s) * SBLK + m
            pltpu.async_copy(ea_hbm.at[pl.ds(bix * BLK, BLK)], evb, esb)
            pltpu.async_copy(ts_hbm.at[sidx.at[sb, m]], rvb, gsb)
            pltpu.async_copy(td_hbm.at[didx.at[sb, m]], dvb, dsb)

        def waitloads(s, sb, m, b):
            evb, rvb, dvb, esb, gsb, dsb, _ = bufs[b]
            bix = (wid * NSUP + s) * SBLK + m
            pltpu.make_async_copy(
                ea_hbm.at[pl.ds(bix * BLK, BLK)], evb, esb).wait()
            pltpu.make_async_copy(ts_hbm.at[sidx.at[sb, m]], rvb, gsb).wait()
            pltpu.make_async_copy(td_hbm.at[didx.at[sb, m]], dvb, dsb).wait()

        def compute(b):
            evb, rvb, dvb, _, _, _, _ = bufs[b]

            def group_body(g, c2):
                rg = g * 16
                rows = rg + lax.iota(jnp.int32, 16)
                amv = amec[0, pl.ds(0, 16)]
                wv = amec[1, pl.ds(0, 16)]
                ead = [plsc.load_gather(
                    evb, [rows, jnp.full((16,), d, jnp.int32)])
                    for d in range(ED)]
                for h in range(H):
                    colA = jnp.full((16,), D + h, jnp.int32)
                    a = plsc.load_gather(rvb, [rows, colA])
                    cD = plsc.load_gather(
                        dvb, [rows, jnp.full((16,), h, jnp.int32)])
                    eA = (ead[0] * wv[h * 4] + ead[1] * wv[h * 4 + 1]
                          + ead[2] * wv[h * 4 + 2] + ead[3] * wv[h * 4 + 3])
                    zb = cD + amv[h]
                    bD = jnp.maximum(zb, 0.2 * zb)
                    z = a + cD + eA
                    z = jnp.maximum(z, 0.2 * z)
                    ex = jnp.exp(z - bD)
                    plsc.store_scatter(rvb, [rows, colA], ex)
                for i in range(16):
                    r = rg + i
                    exv = rvb[r, pl.ds(D, 16)]
                    for h in range(H):
                        s = exv[h]
                        c0 = h * C
                        rvb[r, pl.ds(c0, 16)] = rvb[r, pl.ds(c0, 16)] * s
                        rvb[r, pl.ds(c0 + 16, 16)] = (
                            rvb[r, pl.ds(c0 + 16, 16)] * s)
                return c2

            lax.fori_loop(0, BLK // 16, group_body, 0)

        def scatter(sb, m, b):
            _, rvb, _, _, _, _, ssb = bufs[b]
            pltpu.async_copy(rvb, acc_sh.at[didx.at[sb, m]], ssb, add=True)

        def waitscatter(sb, m, b):
            _, rvb, _, _, _, _, ssb = bufs[b]
            pltpu.make_async_copy(rvb, acc_sh.at[didx.at[sb, m]], ssb).wait()

        bulk_issue(0, 0)

        def super_body(s, carry):
            sb = s % 2
            bulk_wait(s, sb)

            @pl.when(s < NSUP - 1)
            def _():
                bulk_issue(s + 1, 1 - sb)

            issue(s, sb, 0, 0)

            def pair_body(p, c2):
                m0 = 2 * p
                issue(s, sb, m0 + 1, 1)
                waitloads(s, sb, m0, 0)
                compute(0)
                scatter(sb, m0, 0)
                waitscatter(sb, m0, 0)
                issue(s, sb, m0 + 2, 0)
                waitloads(s, sb, m0 + 1, 1)
                compute(1)
                scatter(sb, m0 + 1, 1)
                waitscatter(sb, m0 + 1, 1)
                return c2

            lax.fori_loop(0, (SBLK - 1) // 2, pair_body, 0)
            waitloads(s, sb, SBLK - 1, 0)
            compute(0)
            scatter(sb, SBLK - 1, 0)
            waitscatter(sb, SBLK - 1, 0)
            return carry

        lax.fori_loop(0, NSUP, super_body, 0)
        plsc.subcore_barrier()
        pltpu.sync_copy(acc_sh.at[pl.ds(sid * NROW, NROW)],
                        out_hbm.at[cid, pl.ds(sid * NROW, NROW)])

        @pl.when(sid == NS - 1)
        def _():
            pltpu.sync_copy(acc_sh.at[pl.ds(NS * NROW, NTAIL)],
                            out_hbm.at[cid, pl.ds(NS * NROW, NTAIL)])

    return k(tsrc, adp, ea, src2, dst2, zrows, cvec)


def kernel(x, edge_index, edge_attr, Wp, bp, W0, as0, ad0, ae0, We0, b0, g0,
           be0, W1, as1, ad1, ae1, We1, b1, g1, be1):
    f32 = jnp.float32
    src2 = edge_index[0].reshape(E // (SBLK * SUB), SBLK, SUB)
    dst2 = edge_index[1].reshape(E // (SBLK * SUB), SBLK, SUB)
    zrows = jnp.zeros((NROW, TS), f32)

    def fold(Wf, av):
        return jnp.einsum('dhc,hc->dh', Wf.reshape(-1, H, C), av)

    Wef0 = fold(We0, ae0)
    Wef1 = fold(We1, ae1)
    Wef01 = jnp.concatenate([Wef0, Wef1], axis=1)
    Wcat0 = jnp.concatenate([W0, fold(W0, as0), fold(W0, ad0)], axis=1)
    Wcat1 = jnp.concatenate([W1, fold(W1, as1), fold(W1, ad1)], axis=1)

    easum, emax = _edge_feats(edge_attr, Wef01)
    eamean = easum[0] / E
    eloop0 = eamean @ Wef0
    eloop1 = eamean @ Wef1
    Emax0 = jnp.maximum(emax[0, :H], eloop0)
    Emax1 = jnp.maximum(emax[0, H:], eloop1)

    def cst_of(amax, Emax, eloop):
        return jnp.concatenate(
            [amax[0] + Emax, eloop, jnp.zeros((TD - 2 * H,), f32)]
        ).reshape(1, TD)

    def cvec_of(cst, Wef):
        return jnp.stack([cst[0], Wef.T.reshape(16)])

    Tsrc0, ADp0, amax0 = _prep0(x, Wp, bp, Wcat0)
    cst0 = cst_of(amax0, Emax0, eloop0)
    parts0 = _sc_edge_pass(Tsrc0, ADp0, edge_attr, src2, dst2, zrows,
                           cvec_of(cst0, Wef0))
    prm0 = jnp.stack([b0, g0, be0])
    h1, Tsrc1, ADp1, amax1 = _postprep(parts0, Tsrc0, ADp0, cst0, x, Wp, bp,
                                       prm0, Wcat1)
    cst1 = cst_of(amax1, Emax1, eloop1)
    parts1 = _sc_edge_pass(Tsrc1, ADp1, edge_attr, src2, dst2, zrows,
                           cvec_of(cst1, Wef1))
    prm1 = jnp.stack([b1, g1, be1])
    return _post(parts1, Tsrc1, ADp1, cst1, h1, prm1)


# speedup vs baseline: 91.6718x; 1.0281x over previous
import functools

import jax
import jax.numpy as jnp
from jax import lax
from jax.experimental import pallas as pl
from jax.experimental.pallas import tpu as pltpu
from jax.experimental.pallas import tpu_sc as plsc

N = 10000
E = 320000
DIN = 3
D = 128
H = 4
C = 32
ED = 4

TS = 144
TD = 16
NC = 2
NS = 16
NW = NC * NS
EW = E // NW
SUB = 80
KSUB = 1
SBLK = 25
BLK = SUB * KSUB
NBLK = EW // BLK
NSUP = NBLK // SBLK
NROW = 624
NTAIL = N - NROW * NS


def _node_core(h, wcat_ref, ts_ref, ad_ref, amax_ref):
    t = jnp.dot(h, wcat_ref[...], preferred_element_type=jnp.float32)
    bn = h.shape[0]
    ts_ref[...] = jnp.concatenate(
        [t[:, :D + H], jnp.zeros((bn, TS - D - H), jnp.float32)], axis=1)
    ad_ref[...] = jnp.concatenate(
        [t[:, D + H:D + 2 * H], t[:, D:D + H],
         jnp.zeros((bn, TD - 2 * H), jnp.float32)], axis=1)
    bmax = jnp.max(t[:, D:D + H], axis=0, keepdims=True)

    @pl.when(pl.program_id(0) == 0)
    def _():
        amax_ref[...] = bmax

    @pl.when(pl.program_id(0) != 0)
    def _():
        amax_ref[...] = jnp.maximum(amax_ref[...], bmax)


def _prep0_body(x_ref, wp_ref, bp_ref, wcat_ref, ts_ref, ad_ref, amax_ref):
    h = jnp.dot(x_ref[...], wp_ref[...],
                preferred_element_type=jnp.float32) + bp_ref[...]
    _node_core(h, wcat_ref, ts_ref, ad_ref, amax_ref)


def _prep0(x, Wp, bp, Wcat):
    BN = 2000
    return pl.pallas_call(
        _prep0_body,
        grid=(N // BN,),
        in_specs=[
            pl.BlockSpec((BN, DIN), lambda i: (i, 0)),
            pl.BlockSpec((DIN, D), lambda i: (0, 0)),
            pl.BlockSpec((1, D), lambda i: (0, 0)),
            pl.BlockSpec((D, D + 2 * H), lambda i: (0, 0)),
        ],
        out_specs=[
            pl.BlockSpec((BN, TS), lambda i: (i, 0)),
            pl.BlockSpec((BN, TD), lambda i: (i, 0)),
            pl.BlockSpec((1, H), lambda i: (0, 0)),
        ],
        out_shape=[
            jax.ShapeDtypeStruct((N, TS), jnp.float32),
            jax.ShapeDtypeStruct((N, TD), jnp.float32),
            jax.ShapeDtypeStruct((1, H), jnp.float32),
        ],
    )(x, Wp, bp.reshape(1, D), Wcat)


def _edge_body(ea_ref, wef_ref, sum_ref, max_ref):
    ea = ea_ref[...]
    p = jnp.dot(ea, wef_ref[...], preferred_element_type=jnp.float32)
    bsum = jnp.sum(ea, axis=0, keepdims=True)
    bmax = jnp.max(p, axis=0, keepdims=True)

    @pl.when(pl.program_id(0) == 0)
    def _():
        sum_ref[...] = bsum
        max_ref[...] = bmax

    @pl.when(pl.program_id(0) != 0)
    def _():
        sum_ref[...] += bsum
        max_ref[...] = jnp.maximum(max_ref[...], bmax)


def _edge_feats(ea, Wef01):
    BE = 8000
    return pl.pallas_call(
        _edge_body,
        grid=(E // BE,),
        in_specs=[
            pl.BlockSpec((BE, ED), lambda i: (i, 0)),
            pl.BlockSpec((ED, 2 * H), lambda i: (0, 0)),
        ],
        out_specs=[
            pl.BlockSpec((1, ED), lambda i: (0, 0)),
            pl.BlockSpec((1, 2 * H), lambda i: (0, 0)),
        ],
        out_shape=[
            jax.ShapeDtypeStruct((1, ED), jnp.float32),
            jax.ShapeDtypeStruct((1, 2 * H), jnp.float32),
        ],
    )(ea, Wef01)


def _post_math(p0_ref, p1_ref, ts_ref, ad_ref, cst_ref, res, prm_ref):
    rep = jnp.repeat(jnp.eye(H, dtype=jnp.float32), C, axis=1)
    xs = ts_ref[:, :D]
    ad = ad_ref[...]
    adst = ad[:, :H]
    asrc = ad[:, H:2 * H]
    ame = cst_ref[0, :H]
    eloop = cst_ref[0, H:2 * H]
    zb = adst + ame[None, :]
    bnd = jnp.maximum(zb, 0.2 * zb)
    zs = asrc + adst + eloop[None, :]
    zs = jnp.maximum(zs, 0.2 * zs)
    exs = jnp.exp(zs - bnd)
    raw = (p0_ref[0, :, :D] + p1_ref[0, :, :D]
           + xs * jnp.dot(exs, rep, preferred_element_type=jnp.float32))
    den = p0_ref[0, :, D:D + H] + p1_ref[0, :, D:D + H] + exs + 1e-16
    o = raw * jnp.dot(1.0 / den, rep, preferred_element_type=jnp.float32)
    o = o + prm_ref[0, :][None, :]
    mu = jnp.mean(o, axis=-1, keepdims=True)
    oc = o - mu
    var = jnp.mean(oc * oc, axis=-1, keepdims=True)
    o = oc * jax.lax.rsqrt(var + 1e-5) * prm_ref[1, :][None, :] + prm_ref[2, :][None, :]
    o = jnp.where(o > 0, o, jnp.exp(o) - 1.0)
    return o + res


def _postprep_body(p0_ref, p1_ref, ts_ref, ad_ref, cst_ref, x_ref, wp_ref,
                   bp_ref, prm_ref, wcat_ref, h_ref, ts1_ref, ad1_ref,
                   amax_ref):
    res = jnp.dot(x_ref[...], wp_ref[...],
                  preferred_element_type=jnp.float32) + bp_ref[...]
    h = _post_math(p0_ref, p1_ref, ts_ref, ad_ref, cst_ref, res, prm_ref)
    h_ref[...] = h
    _node_core(h, wcat_ref, ts1_ref, ad1_ref, amax_ref)


def _postprep(parts, Tsrc, ADp, cst, x, Wp, bp, prm, Wcat):
    BN = 2000
    return pl.pallas_call(
        _postprep_body,
        grid=(N // BN,),
        in_specs=[
            pl.BlockSpec((1, BN, TS), lambda i: (0, i, 0)),
            pl.BlockSpec((1, BN, TS), lambda i: (1, i, 0)),
            pl.BlockSpec((BN, TS), lambda i: (i, 0)),
            pl.BlockSpec((BN, TD), lambda i: (i, 0)),
            pl.BlockSpec((1, TD), lambda i: (0, 0)),
            pl.BlockSpec((BN, DIN), lambda i: (i, 0)),
            pl.BlockSpec((DIN, D), lambda i: (0, 0)),
            pl.BlockSpec((1, D), lambda i: (0, 0)),
            pl.BlockSpec((3, D), lambda i: (0, 0)),
            pl.BlockSpec((D, D + 2 * H), lambda i: (0, 0)),
        ],
        out_specs=[
            pl.BlockSpec((BN, D), lambda i: (i, 0)),
            pl.BlockSpec((BN, TS), lambda i: (i, 0)),
            pl.BlockSpec((BN, TD), lambda i: (i, 0)),
            pl.BlockSpec((1, H), lambda i: (0, 0)),
        ],
        out_shape=[
            jax.ShapeDtypeStruct((N, D), jnp.float32),
            jax.ShapeDtypeStruct((N, TS), jnp.float32),
            jax.ShapeDtypeStruct((N, TD), jnp.float32),
            jax.ShapeDtypeStruct((1, H), jnp.float32),
        ],
    )(parts, parts, Tsrc, ADp, cst, x, Wp, bp.reshape(1, D), prm, Wcat)


def _post_final_body(p0_ref, p1_ref, ts_ref, ad_ref, cst_ref, res_ref,
                     prm_ref, o_ref):
    o_ref[...] = _post_math(p0_ref, p1_ref, ts_ref, ad_ref, cst_ref,
                            res_ref[...], prm_ref)


def _post(parts, Tsrc, ADp, cst, res, prm):
    BN = 2000
    return pl.pallas_call(
        _post_final_body,
        grid=(N // BN,),
        in_specs=[
            pl.BlockSpec((1, BN, TS), lambda i: (0, i, 0)),
            pl.BlockSpec((1, BN, TS), lambda i: (1, i, 0)),
            pl.BlockSpec((BN, TS), lambda i: (i, 0)),
            pl.BlockSpec((BN, TD), lambda i: (i, 0)),
            pl.BlockSpec((1, TD), lambda i: (0, 0)),
            pl.BlockSpec((BN, D), lambda i: (i, 0)),
            pl.BlockSpec((3, D), lambda i: (0, 0)),
        ],
        out_specs=pl.BlockSpec((BN, D), lambda i: (i, 0)),
        out_shape=jax.ShapeDtypeStruct((N, D), jnp.float32),
    )(parts, parts, Tsrc, ADp, cst, res, prm)


def _sc_edge_pass(tsrc, adp, ea, src2, dst2, zrows, cvec):
    mesh = plsc.VectorSubcoreMesh(core_axis_name="c", subcore_axis_name="s")

    NSIX = 20
    scratch_types = [
        pltpu.VMEM_SHARED((N, TS), jnp.float32),
        pltpu.VMEM((6, 1, SUB), jnp.int32),
        pltpu.VMEM((6, 1, SUB), jnp.int32),
        pltpu.VMEM((2, BLK, H), jnp.float32),
        pltpu.VMEM((3, BLK, TS), jnp.float32),
        pltpu.VMEM((3, BLK, TD), jnp.float32),
        pltpu.VMEM((2, 16), jnp.float32),
    ] + [pltpu.SemaphoreType.DMA] * 17

    @functools.partial(
        pl.kernel,
        mesh=mesh,
        out_type=jax.ShapeDtypeStruct((NC, N, TS), jnp.float32),
        scratch_types=scratch_types,
        compiler_params=pltpu.CompilerParams(use_tc_tiling_on_sc=False,
                                             needs_layout_passes=False),
    )
    def k(ts_hbm, td_hbm, ea_hbm, s3_hbm, d3_hbm, z_hbm, ame_hbm, out_hbm,
          acc_sh, isv, idv, evs, rvs, dvs, amec,
          i0, i1, i2, i3, i4, i5, e0, e1, g0, g1, g2, d0, d1, d2,
          x0, x1, x2):
        cid = lax.axis_index("c")
        sid = lax.axis_index("s")
        wid = cid * NS + sid
        isem = (i0, i1, i2, i3, i4, i5)
        esem = (e0, e1)
        gsem = (g0, g1, g2)
        dsem = (d0, d1, d2)
        ssem = (x0, x1, x2)

        pltpu.sync_copy(ame_hbm, amec)
        pltpu.sync_copy(z_hbm.at[pl.ds(0, NROW)],
                        acc_sh.at[pl.ds(sid * NROW, NROW)])

        @pl.when(sid == NS - 1)
        def _():
            pltpu.sync_copy(z_hbm.at[pl.ds(0, NTAIL)],
                            acc_sh.at[pl.ds(NS * NROW, NTAIL)])

        plsc.subcore_barrier()

        def idx_issue(j, ki):
            bix = wid * NBLK + j
            pltpu.async_copy(s3_hbm.at[bix], isv.at[ki], isem[ki])
            pltpu.async_copy(d3_hbm.at[bix], idv.at[ki], isem[ki])

        def gath_issue(j, k, ki, ke):
            bix = wid * NBLK + j
            pltpu.make_async_copy(s3_hbm.at[bix], isv.at[ki], isem[ki]).wait()
            pltpu.make_async_copy(d3_hbm.at[bix], idv.at[ki], isem[ki]).wait()
            pltpu.async_copy(ea_hbm.at[pl.ds(bix * BLK, BLK)], evs.at[ke],
                             esem[ke])
            pltpu.async_copy(ts_hbm.at[isv.at[ki, 0]], rvs.at[k], gsem[k])
            pltpu.async_copy(td_hbm.at[idv.at[ki, 0]], dvs.at[k], dsem[k])

        def waitloads(j, k, ki, ke):
            bix = wid * NBLK + j
            pltpu.make_async_copy(ea_hbm.at[pl.ds(bix * BLK, BLK)],
                                  evs.at[ke], esem[ke]).wait()
            pltpu.make_async_copy(ts_hbm.at[isv.at[ki, 0]], rvs.at[k],
                                  gsem[k]).wait()
            pltpu.make_async_copy(td_hbm.at[idv.at[ki, 0]], dvs.at[k],
                                  dsem[k]).wait()

        def compute(k, ke):
            evb = evs.at[ke]
            rvb = rvs.at[k]
            dvb = dvs.at[k]

            def group_body(g, c2):
                rg = g * 16
                rows = rg + lax.iota(jnp.int32, 16)
                amv = amec[0, pl.ds(0, 16)]
                wv = amec[1, pl.ds(0, 16)]
                ead = [plsc.load_gather(
                    evb, [rows, jnp.full((16,), d, jnp.int32)])
                    for d in range(ED)]
                for h in range(H):
                    colA = jnp.full((16,), D + h, jnp.int32)
                    a = plsc.load_gather(rvb, [rows, colA])
                    cD = plsc.load_gather(
                        dvb, [rows, jnp.full((16,), h, jnp.int32)])
                    eA = (ead[0] * wv[h * 4] + ead[1] * wv[h * 4 + 1]
                          + ead[2] * wv[h * 4 + 2] + ead[3] * wv[h * 4 + 3])
                    zb = cD + amv[h]
                    bD = jnp.maximum(zb, 0.2 * zb)
                    z = a + cD + eA
                    z = jnp.maximum(z, 0.2 * z)
                    ex = jnp.exp(z - bD)
                    plsc.store_scatter(rvb, [rows, colA], ex)
                for i in range(16):
                    r = rg + i
                    exv = rvb[r, pl.ds(D, 16)]
                    for h in range(H):
                        s = exv[h]
                        c0 = h * C
                        rvb[r, pl.ds(c0, 16)] = rvb[r, pl.ds(c0, 16)] * s
                        rvb[r, pl.ds(c0 + 16, 16)] = (
                            rvb[r, pl.ds(c0 + 16, 16)] * s)
                return c2

            lax.fori_loop(0, BLK // 16, group_body, 0)

        def cblock(j, k, ki, ke):
            waitloads(j, k, ki, ke)
            compute(k, ke)
            pltpu.async_copy(rvs.at[k], acc_sh.at[idv.at[ki, 0]], ssem[k],
                             add=True)

        def wscat(k, ki):
            pltpu.make_async_copy(rvs.at[k], acc_sh.at[idv.at[ki, 0]],
                                  ssem[k]).wait()

        for j0 in range(4):
            idx_issue(j0, j0)
        gath_issue(0, 0, 0, 0)
        gath_issue(1, 1, 1, 1)

        def six_body(t, carry):
            j = 6 * t
            for p in range(6):
                jp = j + p
                k = p % 3
                cblock(jp, k, p, p % 2)
                if p == 0:
                    @pl.when(jp >= 1)
                    def _():
                        wscat(2, 5)
                else:
                    wscat((p + 2) % 3, p - 1)
                idx_issue(jp + 4, (p + 4) % 6)
                gath_issue(jp + 2, (p + 2) % 3, (p + 2) % 6, p % 2)
            return carry

        lax.fori_loop(0, NSIX, six_body, 0)
        jb = 6 * NSIX
        cblock(jb, 0, 0, 0)
        wscat(2, 5)
        idx_issue(jb + 4, 4)
        gath_issue(jb + 2, 2, 2, 0)
        cblock(jb + 1, 1, 1, 1)
        wscat(0, 0)
        gath_issue(jb + 3, 0, 3, 1)
        cblock(jb + 2, 2, 2, 0)
        wscat(1, 1)
        gath_issue(jb + 4, 1, 4, 0)
        cblock(jb + 3, 0, 3, 1)
        wscat(2, 2)
        cblock(jb + 4, 1, 4, 0)
        wscat(0, 3)
        wscat(1, 4)
        plsc.subcore_barrier()
        pltpu.sync_copy(acc_sh.at[pl.ds(sid * NROW, NROW)],
                        out_hbm.at[cid, pl.ds(sid * NROW, NROW)])

        @pl.when(sid == NS - 1)
        def _():
            pltpu.sync_copy(acc_sh.at[pl.ds(NS * NROW, NTAIL)],
                            out_hbm.at[cid, pl.ds(NS * NROW, NTAIL)])

    return k(tsrc, adp, ea, src2, dst2, zrows, cvec)


def kernel(x, edge_index, edge_attr, Wp, bp, W0, as0, ad0, ae0, We0, b0, g0,
           be0, W1, as1, ad1, ae1, We1, b1, g1, be1):
    f32 = jnp.float32
    src2 = edge_index[0].reshape(E // SUB, 1, SUB)
    dst2 = edge_index[1].reshape(E // SUB, 1, SUB)
    zrows = jnp.zeros((NROW, TS), f32)

    def fold(Wf, av):
        return jnp.einsum('dhc,hc->dh', Wf.reshape(-1, H, C), av)

    Wef0 = fold(We0, ae0)
    Wef1 = fold(We1, ae1)
    Wef01 = jnp.concatenate([Wef0, Wef1], axis=1)
    Wcat0 = jnp.concatenate([W0, fold(W0, as0), fold(W0, ad0)], axis=1)
    Wcat1 = jnp.concatenate([W1, fold(W1, as1), fold(W1, ad1)], axis=1)

    easum, emax = _edge_feats(edge_attr, Wef01)
    eamean = easum[0] / E
    eloop0 = eamean @ Wef0
    eloop1 = eamean @ Wef1
    Emax0 = jnp.maximum(emax[0, :H], eloop0)
    Emax1 = jnp.maximum(emax[0, H:], eloop1)

    def cst_of(amax, Emax, eloop):
        return jnp.concatenate(
            [amax[0] + Emax, eloop, jnp.zeros((TD - 2 * H,), f32)]
        ).reshape(1, TD)

    def cvec_of(cst, Wef):
        return jnp.stack([cst[0], Wef.T.reshape(16)])

    Tsrc0, ADp0, amax0 = _prep0(x, Wp, bp, Wcat0)
    cst0 = cst_of(amax0, Emax0, eloop0)
    parts0 = _sc_edge_pass(Tsrc0, ADp0, edge_attr, src2, dst2, zrows,
                           cvec_of(cst0, Wef0))
    prm0 = jnp.stack([b0, g0, be0])
    h1, Tsrc1, ADp1, amax1 = _postprep(parts0, Tsrc0, ADp0, cst0, x, Wp, bp,
                                       prm0, Wcat1)
    cst1 = cst_of(amax1, Emax1, eloop1)
    parts1 = _sc_edge_pass(Tsrc1, ADp1, edge_attr, src2, dst2, zrows,
                           cvec_of(cst1, Wef1))
    prm1 = jnp.stack([b1, g1, be1])
    return _post(parts1, Tsrc1, ADp1, cst1, h1, prm1)


# speedup vs baseline: 91.6990x; 1.0003x over previous
import functools

import jax
import jax.numpy as jnp
from jax import lax
from jax.experimental import pallas as pl
from jax.experimental.pallas import tpu as pltpu
from jax.experimental.pallas import tpu_sc as plsc

N = 10000
E = 320000
DIN = 3
D = 128
H = 4
C = 32
ED = 4

TS = 144
TD = 16
NC = 2
NS = 16
NW = NC * NS
EW = E // NW
SUB = 80
KSUB = 1
SBLK = 25
BLK = SUB * KSUB
NBLK = EW // BLK
NSUP = NBLK // SBLK
NROW = 624
NTAIL = N - NROW * NS


def _node_core(h, wcat_ref, ts_ref, ad_ref, amax_ref):
    t = jnp.dot(h, wcat_ref[...], preferred_element_type=jnp.float32)
    bn = h.shape[0]
    ts_ref[...] = jnp.concatenate(
        [t[:, :D + H], jnp.zeros((bn, TS - D - H), jnp.float32)], axis=1)
    ad_ref[...] = jnp.concatenate(
        [t[:, D + H:D + 2 * H], t[:, D:D + H],
         jnp.zeros((bn, TD - 2 * H), jnp.float32)], axis=1)
    bmax = jnp.max(t[:, D:D + H], axis=0, keepdims=True)

    @pl.when(pl.program_id(0) == 0)
    def _():
        amax_ref[...] = bmax

    @pl.when(pl.program_id(0) != 0)
    def _():
        amax_ref[...] = jnp.maximum(amax_ref[...], bmax)


def _prep0_body(x_ref, wp_ref, bp_ref, wcat_ref, ts_ref, ad_ref, amax_ref):
    h = jnp.dot(x_ref[...], wp_ref[...],
                preferred_element_type=jnp.float32) + bp_ref[...]
    _node_core(h, wcat_ref, ts_ref, ad_ref, amax_ref)


def _prep0(x, Wp, bp, Wcat):
    BN = 2000
    return pl.pallas_call(
        _prep0_body,
        grid=(N // BN,),
        in_specs=[
            pl.BlockSpec((BN, DIN), lambda i: (i, 0)),
            pl.BlockSpec((DIN, D), lambda i: (0, 0)),
            pl.BlockSpec((1, D), lambda i: (0, 0)),
            pl.BlockSpec((D, D + 2 * H), lambda i: (0, 0)),
        ],
        out_specs=[
            pl.BlockSpec((BN, TS), lambda i: (i, 0)),
            pl.BlockSpec((BN, TD), lambda i: (i, 0)),
            pl.BlockSpec((1, H), lambda i: (0, 0)),
        ],
        out_shape=[
            jax.ShapeDtypeStruct((N, TS), jnp.float32),
            jax.ShapeDtypeStruct((N, TD), jnp.float32),
            jax.ShapeDtypeStruct((1, H), jnp.float32),
        ],
    )(x, Wp, bp.reshape(1, D), Wcat)


def _edge_body(ea_ref, wef_ref, sum_ref, max_ref):
    ea = ea_ref[...]
    p = jnp.dot(ea, wef_ref[...], preferred_element_type=jnp.float32)
    bsum = jnp.sum(ea, axis=0, keepdims=True)
    bmax = jnp.max(p, axis=0, keepdims=True)

    @pl.when(pl.program_id(0) == 0)
    def _():
        sum_ref[...] = bsum
        max_ref[...] = bmax

    @pl.when(pl.program_id(0) != 0)
    def _():
        sum_ref[...] += bsum
        max_ref[...] = jnp.maximum(max_ref[...], bmax)


def _edge_feats(ea, Wef01):
    BE = 8000
    return pl.pallas_call(
        _edge_body,
        grid=(E // BE,),
        in_specs=[
            pl.BlockSpec((BE, ED), lambda i: (i, 0)),
            pl.BlockSpec((ED, 2 * H), lambda i: (0, 0)),
        ],
        out_specs=[
            pl.BlockSpec((1, ED), lambda i: (0, 0)),
            pl.BlockSpec((1, 2 * H), lambda i: (0, 0)),
        ],
        out_shape=[
            jax.ShapeDtypeStruct((1, ED), jnp.float32),
            jax.ShapeDtypeStruct((1, 2 * H), jnp.float32),
        ],
    )(ea, Wef01)


def _post_math(p0_ref, p1_ref, ts_ref, ad_ref, cst_ref, res, prm_ref):
    rep = jnp.repeat(jnp.eye(H, dtype=jnp.float32), C, axis=1)
    xs = ts_ref[:, :D]
    ad = ad_ref[...]
    adst = ad[:, :H]
    asrc = ad[:, H:2 * H]
    ame = cst_ref[0, :H]
    eloop = cst_ref[0, H:2 * H]
    zb = adst + ame[None, :]
    bnd = jnp.maximum(zb, 0.2 * zb)
    zs = asrc + adst + eloop[None, :]
    zs = jnp.maximum(zs, 0.2 * zs)
    exs = jnp.exp(zs - bnd)
    raw = (p0_ref[0, :, :D] + p1_ref[0, :, :D]
           + xs * jnp.dot(exs, rep, preferred_element_type=jnp.float32))
    den = p0_ref[0, :, D:D + H] + p1_ref[0, :, D:D + H] + exs + 1e-16
    o = raw * jnp.dot(1.0 / den, rep, preferred_element_type=jnp.float32)
    o = o + prm_ref[0, :][None, :]
    mu = jnp.mean(o, axis=-1, keepdims=True)
    oc = o - mu
    var = jnp.mean(oc * oc, axis=-1, keepdims=True)
    o = oc * jax.lax.rsqrt(var + 1e-5) * prm_ref[1, :][None, :] + prm_ref[2, :][None, :]
    o = jnp.where(o > 0, o, jnp.exp(o) - 1.0)
    return o + res


def _postprep_body(p0_ref, p1_ref, ts_ref, ad_ref, cst_ref, x_ref, wp_ref,
                   bp_ref, prm_ref, wcat_ref, h_ref, ts1_ref, ad1_ref,
                   amax_ref):
    res = jnp.dot(x_ref[...], wp_ref[...],
                  preferred_element_type=jnp.float32) + bp_ref[...]
    h = _post_math(p0_ref, p1_ref, ts_ref, ad_ref, cst_ref, res, prm_ref)
    h_ref[...] = h
    _node_core(h, wcat_ref, ts1_ref, ad1_ref, amax_ref)


def _postprep(parts, Tsrc, ADp, cst, x, Wp, bp, prm, Wcat):
    BN = 2000
    return pl.pallas_call(
        _postprep_body,
        grid=(N // BN,),
        in_specs=[
            pl.BlockSpec((1, BN, TS), lambda i: (0, i, 0)),
            pl.BlockSpec((1, BN, TS), lambda i: (1, i, 0)),
            pl.BlockSpec((BN, TS), lambda i: (i, 0)),
            pl.BlockSpec((BN, TD), lambda i: (i, 0)),
            pl.BlockSpec((1, TD), lambda i: (0, 0)),
            pl.BlockSpec((BN, DIN), lambda i: (i, 0)),
            pl.BlockSpec((DIN, D), lambda i: (0, 0)),
            pl.BlockSpec((1, D), lambda i: (0, 0)),
            pl.BlockSpec((3, D), lambda i: (0, 0)),
            pl.BlockSpec((D, D + 2 * H), lambda i: (0, 0)),
        ],
        out_specs=[
            pl.BlockSpec((BN, D), lambda i: (i, 0)),
            pl.BlockSpec((BN, TS), lambda i: (i, 0)),
            pl.BlockSpec((BN, TD), lambda i: (i, 0)),
            pl.BlockSpec((1, H), lambda i: (0, 0)),
        ],
        out_shape=[
            jax.ShapeDtypeStruct((N, D), jnp.float32),
            jax.ShapeDtypeStruct((N, TS), jnp.float32),
            jax.ShapeDtypeStruct((N, TD), jnp.float32),
            jax.ShapeDtypeStruct((1, H), jnp.float32),
        ],
    )(parts, parts, Tsrc, ADp, cst, x, Wp, bp.reshape(1, D), prm, Wcat)


def _post_final_body(p0_ref, p1_ref, ts_ref, ad_ref, cst_ref, res_ref,
                     prm_ref, o_ref):
    o_ref[...] = _post_math(p0_ref, p1_ref, ts_ref, ad_ref, cst_ref,
                            res_ref[...], prm_ref)


def _post(parts, Tsrc, ADp, cst, res, prm):
    BN = 2000
    return pl.pallas_call(
        _post_final_body,
        grid=(N // BN,),
        in_specs=[
            pl.BlockSpec((1, BN, TS), lambda i: (0, i, 0)),
            pl.BlockSpec((1, BN, TS), lambda i: (1, i, 0)),
            pl.BlockSpec((BN, TS), lambda i: (i, 0)),
            pl.BlockSpec((BN, TD), lambda i: (i, 0)),
            pl.BlockSpec((1, TD), lambda i: (0, 0)),
            pl.BlockSpec((BN, D), lambda i: (i, 0)),
            pl.BlockSpec((3, D), lambda i: (0, 0)),
        ],
        out_specs=pl.BlockSpec((BN, D), lambda i: (i, 0)),
        out_shape=jax.ShapeDtypeStruct((N, D), jnp.float32),
    )(parts, parts, Tsrc, ADp, cst, res, prm)


def _sc_edge_pass(tsrc, adp, ea, src2, dst2, zrows, cvec):
    mesh = plsc.VectorSubcoreMesh(core_axis_name="c", subcore_axis_name="s")

    NSIX = 20
    scratch_types = [
        pltpu.VMEM_SHARED((N, TS), jnp.float32),
        pltpu.VMEM((6, 1, SUB), jnp.int32),
        pltpu.VMEM((6, 1, SUB), jnp.int32),
        pltpu.VMEM((2, BLK, H), jnp.float32),
        pltpu.VMEM((3, BLK, TS), jnp.float32),
        pltpu.VMEM((3, BLK, TD), jnp.float32),
        pltpu.VMEM((2, 16), jnp.float32),
    ] + [pltpu.SemaphoreType.DMA] * 17

    @functools.partial(
        pl.kernel,
        mesh=mesh,
        out_type=jax.ShapeDtypeStruct((NC, N, TS), jnp.float32),
        scratch_types=scratch_types,
        compiler_params=pltpu.CompilerParams(use_tc_tiling_on_sc=False,
                                             needs_layout_passes=False,
                                             disable_bounds_checks=True,
                                             skip_device_barrier=True),
    )
    def k(ts_hbm, td_hbm, ea_hbm, s3_hbm, d3_hbm, z_hbm, ame_hbm, out_hbm,
          acc_sh, isv, idv, evs, rvs, dvs, amec,
          i0, i1, i2, i3, i4, i5, e0, e1, g0, g1, g2, d0, d1, d2,
          x0, x1, x2):
        cid = lax.axis_index("c")
        sid = lax.axis_index("s")
        wid = cid * NS + sid
        isem = (i0, i1, i2, i3, i4, i5)
        esem = (e0, e1)
        gsem = (g0, g1, g2)
        dsem = (d0, d1, d2)
        ssem = (x0, x1, x2)

        pltpu.sync_copy(ame_hbm, amec)
        pltpu.sync_copy(z_hbm.at[pl.ds(0, NROW)],
                        acc_sh.at[pl.ds(sid * NROW, NROW)])

        @pl.when(sid == NS - 1)
        def _():
            pltpu.sync_copy(z_hbm.at[pl.ds(0, NTAIL)],
                            acc_sh.at[pl.ds(NS * NROW, NTAIL)])

        plsc.subcore_barrier()

        def idx_issue(j, ki):
            bix = wid * NBLK + j
            pltpu.async_copy(s3_hbm.at[bix], isv.at[ki], isem[ki])
            pltpu.async_copy(d3_hbm.at[bix], idv.at[ki], isem[ki])

        def gath_issue(j, k, ki, ke):
            bix = wid * NBLK + j
            pltpu.make_async_copy(s3_hbm.at[bix], isv.at[ki], isem[ki]).wait()
            pltpu.make_async_copy(d3_hbm.at[bix], idv.at[ki], isem[ki]).wait()
            pltpu.async_copy(ea_hbm.at[pl.ds(bix * BLK, BLK)], evs.at[ke],
                             esem[ke])
            pltpu.async_copy(ts_hbm.at[isv.at[ki, 0]], rvs.at[k], gsem[k])
            pltpu.async_copy(td_hbm.at[idv.at[ki, 0]], dvs.at[k], dsem[k])

        def waitloads(j, k, ki, ke):
            bix = wid * NBLK + j
            pltpu.make_async_copy(ea_hbm.at[pl.ds(bix * BLK, BLK)],
                                  evs.at[ke], esem[ke]).wait()
            pltpu.make_async_copy(ts_hbm.at[isv.at[ki, 0]], rvs.at[k],
                                  gsem[k]).wait()
            pltpu.make_async_copy(td_hbm.at[idv.at[ki, 0]], dvs.at[k],
                                  dsem[k]).wait()

        def compute(k, ke):
            evb = evs.at[ke]
            rvb = rvs.at[k]
            dvb = dvs.at[k]

            def group_body(g, c2):
                rg = g * 16
                rows = rg + lax.iota(jnp.int32, 16)
                amv = amec[0, pl.ds(0, 16)]
                wv = amec[1, pl.ds(0, 16)]
                ead = [plsc.load_gather(
                    evb, [rows, jnp.full((16,), d, jnp.int32)])
                    for d in range(ED)]
                for h in range(H):
                    colA = jnp.full((16,), D + h, jnp.int32)
                    a = plsc.load_gather(rvb, [rows, colA])
                    cD = plsc.load_gather(
                        dvb, [rows, jnp.full((16,), h, jnp.int32)])
                    eA = (ead[0] * wv[h * 4] + ead[1] * wv[h * 4 + 1]
                          + ead[2] * wv[h * 4 + 2] + ead[3] * wv[h * 4 + 3])
                    zb = cD + amv[h]
                    bD = jnp.maximum(zb, 0.2 * zb)
                    z = a + cD + eA
                    z = jnp.maximum(z, 0.2 * z)
                    ex = jnp.exp(z - bD)
                    plsc.store_scatter(rvb, [rows, colA], ex)
                for i in range(16):
                    r = rg + i
                    exv = rvb[r, pl.ds(D, 16)]
                    for h in range(H):
                        s = exv[h]
                        c0 = h * C
                        rvb[r, pl.ds(c0, 16)] = rvb[r, pl.ds(c0, 16)] * s
                        rvb[r, pl.ds(c0 + 16, 16)] = (
                            rvb[r, pl.ds(c0 + 16, 16)] * s)
                return c2

            lax.fori_loop(0, BLK // 16, group_body, 0)

        def cblock(j, k, ki, ke):
            waitloads(j, k, ki, ke)
            compute(k, ke)
            pltpu.async_copy(rvs.at[k], acc_sh.at[idv.at[ki, 0]], ssem[k],
                             add=True)

        def wscat(k, ki):
            pltpu.make_async_copy(rvs.at[k], acc_sh.at[idv.at[ki, 0]],
                                  ssem[k]).wait()

        for j0 in range(4):
            idx_issue(j0, j0)
        gath_issue(0, 0, 0, 0)
        gath_issue(1, 1, 1, 1)

        def six_body(t, carry):
            j = 6 * t
            for p in range(6):
                jp = j + p
                k = p % 3
                cblock(jp, k, p, p % 2)
                if p == 0:
                    @pl.when(jp >= 1)
                    def _():
                        wscat(2, 5)
                else:
                    wscat((p + 2) % 3, p - 1)
                idx_issue(jp + 4, (p + 4) % 6)
                gath_issue(jp + 2, (p + 2) % 3, (p + 2) % 6, p % 2)
            return carry

        lax.fori_loop(0, NSIX, six_body, 0)
        jb = 6 * NSIX
        cblock(jb, 0, 0, 0)
        wscat(2, 5)
        idx_issue(jb + 4, 4)
        gath_issue(jb + 2, 2, 2, 0)
        cblock(jb + 1, 1, 1, 1)
        wscat(0, 0)
        gath_issue(jb + 3, 0, 3, 1)
        cblock(jb + 2, 2, 2, 0)
        wscat(1, 1)
        gath_issue(jb + 4, 1, 4, 0)
        cblock(jb + 3, 0, 3, 1)
        wscat(2, 2)
        cblock(jb + 4, 1, 4, 0)
        wscat(0, 3)
        wscat(1, 4)
        plsc.subcore_barrier()
        pltpu.sync_copy(acc_sh.at[pl.ds(sid * NROW, NROW)],
                        out_hbm.at[cid, pl.ds(sid * NROW, NROW)])

        @pl.when(sid == NS - 1)
        def _():
            pltpu.sync_copy(acc_sh.at[pl.ds(NS * NROW, NTAIL)],
                            out_hbm.at[cid, pl.ds(NS * NROW, NTAIL)])

    return k(tsrc, adp, ea, src2, dst2, zrows, cvec)


def kernel(x, edge_index, edge_attr, Wp, bp, W0, as0, ad0, ae0, We0, b0, g0,
           be0, W1, as1, ad1, ae1, We1, b1, g1, be1):
    f32 = jnp.float32
    src2 = edge_index[0].reshape(E // SUB, 1, SUB)
    dst2 = edge_index[1].reshape(E // SUB, 1, SUB)
    zrows = jnp.zeros((NROW, TS), f32)

    def fold(Wf, av):
        return jnp.einsum('dhc,hc->dh', Wf.reshape(-1, H, C), av)

    Wef0 = fold(We0, ae0)
    Wef1 = fold(We1, ae1)
    Wef01 = jnp.concatenate([Wef0, Wef1], axis=1)
    Wcat0 = jnp.concatenate([W0, fold(W0, as0), fold(W0, ad0)], axis=1)
    Wcat1 = jnp.concatenate([W1, fold(W1, as1), fold(W1, ad1)], axis=1)

    easum, emax = _edge_feats(edge_attr, Wef01)
    eamean = easum[0] / E
    eloop0 = eamean @ Wef0
    eloop1 = eamean @ Wef1
    Emax0 = jnp.maximum(emax[0, :H], eloop0)
    Emax1 = jnp.maximum(emax[0, H:], eloop1)

    def cst_of(amax, Emax, eloop):
        return jnp.concatenate(
            [amax[0] + Emax, eloop, jnp.zeros((TD - 2 * H,), f32)]
        ).reshape(1, TD)

    def cvec_of(cst, Wef):
        return jnp.stack([cst[0], Wef.T.reshape(16)])

    Tsrc0, ADp0, amax0 = _prep0(x, Wp, bp, Wcat0)
    cst0 = cst_of(amax0, Emax0, eloop0)
    parts0 = _sc_edge_pass(Tsrc0, ADp0, edge_attr, src2, dst2, zrows,
                           cvec_of(cst0, Wef0))
    prm0 = jnp.stack([b0, g0, be0])
    h1, Tsrc1, ADp1, amax1 = _postprep(parts0, Tsrc0, ADp0, cst0, x, Wp, bp,
                                       prm0, Wcat1)
    cst1 = cst_of(amax1, Emax1, eloop1)
    parts1 = _sc_edge_pass(Tsrc1, ADp1, edge_attr, src2, dst2, zrows,
                           cvec_of(cst1, Wef1))
    prm1 = jnp.stack([b1, g1, be1])
    return _post(parts1, Tsrc1, ADp1, cst1, h1, prm1)
